# baseline (speedup 1.0000x reference)
import functools

import jax
import jax.numpy as jnp
from jax import lax
from jax.experimental import pallas as pl
from jax.experimental.pallas import tpu as pltpu
from jax.experimental.pallas import tpu_sc as plsc

N_NODES_C = 10000
N_EDGES_C = 320000
HID_C = 128
HALF = 64
NUM_BOND_C = 24
NUM_ATOM_C = 100

NC = 2
NS = 16
NW = NC * NS
EPW = N_EDGES_C // NW
CHUNK = 80
N_CHUNKS = EPW // CHUNK
NBUF = 5
N_ROUNDS = N_CHUNKS // NBUF
RTA = 624
TAIL = N_NODES_C - NS * RTA

_MESH = plsc.VectorSubcoreMesh(core_axis_name="c", subcore_axis_name="s")


def _silu(x):
    return x * (1.0 / (1.0 + jnp.exp(-x)))


def _mul_rows(rows, other):
    def mulrow(r, cc):
        for j in range(HALF // 16):
            sl = pl.ds(j * 16, 16)
            rows[r, sl] = rows[r, sl] * other[r, sl]
        return cc

    lax.fori_loop(0, CHUNK, mulrow, 0)


def _node_embed_body(at_ref, rf_ref, pf_ref, tab_ref, wf_ref, zlo_ref, zhi_ref):
    oh = (at_ref[...] == lax.broadcasted_iota(jnp.int32, (at_ref.shape[0], NUM_ATOM_C), 1)
          ).astype(jnp.float32)
    emb = jnp.dot(oh, tab_ref[...], preferred_element_type=jnp.float32)
    fr = jnp.dot(rf_ref[...], wf_ref[...], preferred_element_type=jnp.float32)
    fp = jnp.dot(pf_ref[...], wf_ref[...], preferred_element_type=jnp.float32)
    zlo_ref[...] = emb + fr
    zhi_ref[...] = fp - fr


def _node_embed(atom_type, r_feat, p_feat, atom_tab, W_feat):
    nb = 1000
    grid = N_NODES_C // nb
    return pl.pallas_call(
        _node_embed_body,
        grid=(grid,),
        in_specs=[
            pl.BlockSpec((nb, 1), lambda i: (i, 0)),
            pl.BlockSpec((nb, 128), lambda i: (i, 0)),
            pl.BlockSpec((nb, 128), lambda i: (i, 0)),
            pl.BlockSpec((NUM_ATOM_C, HALF), lambda i: (0, 0)),
            pl.BlockSpec((128, HALF), lambda i: (0, 0)),
        ],
        out_specs=[pl.BlockSpec((nb, HALF), lambda i: (i, 0))] * 2,
        out_shape=[jax.ShapeDtypeStruct((N_NODES_C, HALF), jnp.float32)] * 2,
    )(atom_type.reshape(N_NODES_C, 1).astype(jnp.int32), r_feat, p_feat,
      atom_tab, W_feat)


def _d2_sc_body(px_h, py_h, pz_h, src_h, dst_h, d2_h,
                px_v, py_v, pz_v, src_v, dst_v, out_v):
    wid = lax.axis_index("s") * NC + lax.axis_index("c")
    base = wid * EPW
    pltpu.sync_copy(px_h, px_v)
    pltpu.sync_copy(py_h, py_v)
    pltpu.sync_copy(pz_h, pz_v)
    pltpu.sync_copy(src_h.at[pl.ds(base, EPW)], src_v)
    pltpu.sync_copy(dst_h.at[pl.ds(base, EPW)], dst_v)

    def body(i, carry):
        sl = pl.ds(i * 16, 16)
        s = src_v[sl]
        t = dst_v[sl]
        dx = plsc.load_gather(px_v, [t]) - plsc.load_gather(px_v, [s])
        dy = plsc.load_gather(py_v, [t]) - plsc.load_gather(py_v, [s])
        dz = plsc.load_gather(pz_v, [t]) - plsc.load_gather(pz_v, [s])
        out_v[sl] = dx * dx + dy * dy + dz * dz
        return carry

    lax.fori_loop(0, EPW // 16, body, 0)
    pltpu.sync_copy(out_v, d2_h.at[pl.ds(base, EPW)])


def _edge_d2(pos, src, dst):
    k = functools.partial(
        pl.kernel,
        out_type=jax.ShapeDtypeStruct((N_EDGES_C,), jnp.float32),
        mesh=_MESH,
        scratch_types=[
            pltpu.VMEM((N_NODES_C,), jnp.float32),
            pltpu.VMEM((N_NODES_C,), jnp.float32),
            pltpu.VMEM((N_NODES_C,), jnp.float32),
            pltpu.VMEM((EPW,), jnp.int32),
            pltpu.VMEM((EPW,), jnp.int32),
            pltpu.VMEM((EPW,), jnp.float32),
        ],
        compiler_params=pltpu.CompilerParams(needs_layout_passes=False),
    )(_d2_sc_body)
    pos_t = pos.T
    return k(pos_t[0], pos_t[1], pos_t[2], src, dst)


def _edge_attr_body(d2_ref, bt_ref,
                    we1g_ref, be1g_ref, we2g_ref, be2g_ref, bondg_ref,
                    we1l_ref, be1l_ref, we2l_ref, be2l_ref, bondl_ref,
                    eag_ref, eal_ref):
    d = jnp.sqrt(d2_ref[...])
    oh = (bt_ref[...] == lax.broadcasted_iota(jnp.int32, (bt_ref.shape[0], NUM_BOND_C), 1)
          ).astype(jnp.float32)

    def branch(we1, be1, we2, be2, bond, out_ref):
        e1 = _silu(d * we1[...] + be1[...])
        e = jnp.dot(e1, we2[...], preferred_element_type=jnp.float32) + be2[...]
        bemb = jnp.dot(oh, bond[...], preferred_element_type=jnp.float32)
        out_ref[...] = e * bemb

    branch(we1g_ref, be1g_ref, we2g_ref, be2g_ref, bondg_ref, eag_ref)
    branch(we1l_ref, be1l_ref, we2l_ref, be2l_ref, bondl_ref, eal_ref)


def _edge_attr(d2, bond_type, We1_g, be1_g, We2_g, be2_g, bond_g,
               We1_l, be1_l, We2_l, be2_l, bond_l):
    eb = 2560
    grid = N_EDGES_C // eb
    w_spec = lambda shape: pl.BlockSpec(shape, lambda i: (0, 0))
    return pl.pallas_call(
        _edge_attr_body,
        grid=(grid,),
        in_specs=[
            pl.BlockSpec((eb, 1), lambda i: (i, 0)),
            pl.BlockSpec((eb, 1), lambda i: (i, 0)),
            w_spec((1, HID_C)), w_spec((1, HID_C)), w_spec((HID_C, HID_C)),
            w_spec((1, HID_C)), w_spec((NUM_BOND_C, HID_C)),
            w_spec((1, HID_C)), w_spec((1, HID_C)), w_spec((HID_C, HID_C)),
            w_spec((1, HID_C)), w_spec((NUM_BOND_C, HID_C)),
        ],
        out_specs=[pl.BlockSpec((eb, HID_C), lambda i: (i, 0))] * 2,
        out_shape=[jax.ShapeDtypeStruct((N_EDGES_C, HID_C), jnp.float32)] * 2,
    )(d2.reshape(N_EDGES_C, 1), bond_type.reshape(N_EDGES_C, 1).astype(jnp.int32),
      We1_g, be1_g.reshape(1, HID_C), We2_g, be2_g.reshape(1, HID_C), bond_g,
      We1_l, be1_l.reshape(1, HID_C), We2_l, be2_l.reshape(1, HID_C), bond_l)


GB = 25
EB = GB * CHUNK
NG = N_CHUNKS // GB


def _filters_body(eag_ref, eal_ref,
                  wf1g_ref, bf1g_ref, wf2g_ref, bf2g_ref,
                  wf1l_ref, bf1l_ref, wf2l_ref, bf2l_ref,
                  fg_lo_ref, fg_hi_ref, fl_lo_ref, fl_hi_ref):
    def branch(ea_ref, wf1, bf1, wf2, bf2, lo_ref, hi_ref):
        t = _silu(jnp.dot(ea_ref[...], wf1[...], preferred_element_type=jnp.float32)
                  + bf1[...])
        f = jnp.dot(t, wf2[...], preferred_element_type=jnp.float32) + bf2[...]
        lo_ref[...] = f[:, :HALF].reshape(1, GB, CHUNK, HALF)
        hi_ref[...] = f[:, HALF:].reshape(1, GB, CHUNK, HALF)

    branch(eag_ref, wf1g_ref, bf1g_ref, wf2g_ref, bf2g_ref, fg_lo_ref, fg_hi_ref)
    branch(eal_ref, wf1l_ref, bf1l_ref, wf2l_ref, bf2l_ref, fl_lo_ref, fl_hi_ref)


def _filters(ea_g, ea_l, wf1_g, bf1_g, wf2_g, bf2_g, wf1_l, bf1_l, wf2_l, bf2_l):
    w_spec = lambda shape: pl.BlockSpec(shape, lambda w, g: (0, 0))
    e_spec = pl.BlockSpec((EB, HID_C), lambda w, g: (NG * w + g, 0))
    f_spec = pl.BlockSpec((1, GB, CHUNK, HALF), lambda w, g: (w, g, 0, 0))
    return pl.pallas_call(
        _filters_body,
        grid=(NW, NG),
        in_specs=[
            e_spec, e_spec,
            w_spec((HID_C, HID_C)), w_spec((1, HID_C)),
            w_spec((HID_C, HID_C)), w_spec((1, HID_C)),
            w_spec((HID_C, HID_C)), w_spec((1, HID_C)),
            w_spec((HID_C, HID_C)), w_spec((1, HID_C)),
        ],
        out_specs=[f_spec] * 4,
        out_shape=[jax.ShapeDtypeStruct((NW, N_CHUNKS, CHUNK, HALF),
                                        jnp.float32)] * 4,
    )(ea_g, ea_l,
      wf1_g, bf1_g.reshape(1, HID_C), wf2_g, bf2_g.reshape(1, HID_C),
      wf1_l, bf1_l.reshape(1, HID_C), wf2_l, bf2_l.reshape(1, HID_C))


def _gms_body(hlo_h, hhi_h, flo_h, fhi_h, src_h, dst_h, zeros_h, out_h,
              src_v, dst_v, rows_v, filt_v, agg_sh, *sems):
    gf_sem = sems[:NBUF]
    s_sem = sems[NBUF:]
    c = lax.axis_index("c")
    s = lax.axis_index("s")
    wid = s * NC + c

    pltpu.sync_copy(src_h.at[wid], src_v)
    pltpu.sync_copy(dst_h.at[wid], dst_v)

    for half, (h_h, f_h) in enumerate(((hlo_h, flo_h), (hhi_h, fhi_h))):
        def issue_gf(j, b):
            pltpu.async_copy(h_h.at[src_v.at[j]], rows_v.at[b], gf_sem[b])
            pltpu.async_copy(f_h.at[wid, j], filt_v.at[b], gf_sem[b])

        def wait_gf(b):
            pltpu.make_async_copy(h_h.at[src_v.at[0]], rows_v.at[b],
                                  gf_sem[b]).wait()
            pltpu.make_async_copy(f_h.at[wid, 0], filt_v.at[b], gf_sem[b]).wait()

        def wait_scatter(b):
            pltpu.make_async_copy(rows_v.at[b], agg_sh.at[dst_v.at[0]],
                                  s_sem[b]).wait()

        for b in range(NBUF - 1):
            issue_gf(b, b)

        pltpu.sync_copy(zeros_h, agg_sh.at[pl.ds(s * RTA, RTA)])

        @pl.when(s == 0)
        def _zero_tail():
            pltpu.sync_copy(zeros_h.at[pl.ds(0, TAIL)],
                            agg_sh.at[pl.ds(NS * RTA, TAIL)])

        plsc.subcore_barrier()

        def round_body(r, carry):
            i0 = r * NBUF
            for b in range(NBUF):
                i = i0 + b
                bj = (b + NBUF - 1) % NBUF
                j = i + NBUF - 1

                @pl.when(j < N_CHUNKS)
                def _prefetch():
                    @pl.when(j >= NBUF)
                    def _drain():
                        wait_scatter(bj)

                    issue_gf(j, bj)

                wait_gf(b)
                _mul_rows(rows_v.at[b], filt_v.at[b])
                pltpu.async_copy(rows_v.at[b], agg_sh.at[dst_v.at[i]], s_sem[b],
                                 add=True)
            return carry

        lax.fori_loop(0, N_ROUNDS, round_body, 0)
        for b in range(NBUF):
            wait_scatter(b)
        plsc.subcore_barrier()
        pltpu.sync_copy(agg_sh.at[pl.ds(s * RTA, RTA)],
                        out_h.at[c, half, pl.ds(s * RTA, RTA)])

        @pl.when(s == 0)
        def _copy_tail():
            pltpu.sync_copy(agg_sh.at[pl.ds(NS * RTA, TAIL)],
                            out_h.at[c, half, pl.ds(NS * RTA, TAIL)])

        plsc.subcore_barrier()


def _gather_mul_scatter(h_lo, h_hi, f_lo4, f_hi4, src3, dst3, zeros_block):
    k = functools.partial(
        pl.kernel,
        out_type=jax.ShapeDtypeStruct((NC, 2, N_NODES_C, HALF), jnp.float32),
        mesh=_MESH,
        scratch_types=[
            pltpu.VMEM((N_CHUNKS, CHUNK), jnp.int32),
            pltpu.VMEM((N_CHUNKS, CHUNK), jnp.int32),
            pltpu.VMEM((NBUF, CHUNK, HALF), jnp.float32),
            pltpu.VMEM((NBUF, CHUNK, HALF), jnp.float32),
            pltpu.VMEM_SHARED((N_NODES_C, HALF), jnp.float32),
        ] + [pltpu.SemaphoreType.DMA] * (2 * NBUF),
        compiler_params=pltpu.CompilerParams(use_tc_tiling_on_sc=False),
    )(_gms_body)
    return k(h_lo, h_hi, f_lo4, f_hi4, src3, dst3, zeros_block)


def _update_body(hlg_ref, hhg_ref, hll_ref, hhl_ref, ag_ref, al_ref,
                 wug_ref, bug_ref, wul_ref, bul_ref,
                 olg_ref, ohg_ref, oll_ref, ohl_ref):
    def branch(hlo_ref, hhi_ref, a_ref, wu, bu, olo_ref, ohi_ref):
        agg = jnp.concatenate([a_ref[0, 0] + a_ref[1, 0],
                               a_ref[0, 1] + a_ref[1, 1]], axis=1)
        x = _silu(jnp.dot(agg, wu[...], preferred_element_type=jnp.float32)
                  + bu[...])
        olo_ref[...] = hlo_ref[...] + x[:, :HALF]
        ohi_ref[...] = hhi_ref[...] + x[:, HALF:]

    branch(hlg_ref, hhg_ref, ag_ref, wug_ref, bug_ref, olg_ref, ohg_ref)
    branch(hll_ref, hhl_ref, al_ref, wul_ref, bul_ref, oll_ref, ohl_ref)


def _node_update(hs_g, hs_l, agg_g, agg_l, wu_g, bu_g, wu_l, bu_l):
    nb = 1000
    grid = N_NODES_C // nb
    w_spec = lambda shape: pl.BlockSpec(shape, lambda i: tuple([0] * len(shape)))
    h_spec = pl.BlockSpec((nb, HALF), lambda i: (i, 0))
    a_spec = pl.BlockSpec((NC, 2, nb, HALF), lambda i: (0, 0, i, 0))
    out = pl.pallas_call(
        _update_body,
        grid=(grid,),
        in_specs=[h_spec, h_spec, h_spec, h_spec, a_spec, a_spec,
                  w_spec((HID_C, HID_C)), w_spec((1, HID_C)),
                  w_spec((HID_C, HID_C)), w_spec((1, HID_C))],
        out_specs=[h_spec] * 4,
        out_shape=[jax.ShapeDtypeStruct((N_NODES_C, HALF), jnp.float32)] * 4,
    )(hs_g[0], hs_g[1], hs_l[0], hs_l[1], agg_g, agg_l,
      wu_g, bu_g.reshape(1, HID_C), wu_l, bu_l.reshape(1, HID_C))
    return (out[0], out[1]), (out[2], out[3])


def _pair_body(hlg_h, hhg_h, hll_h, hhl_h, src_h, dst_h,
               olg_h, ohg_h, oll_h, ohl_h,
               src_v, dst_v, rs_v, rd_v, *sems):
    gf_sem = sems[:NBUF]
    w_sem = sems[NBUF:]
    wid = lax.axis_index("s") * NC + lax.axis_index("c")

    pltpu.sync_copy(src_h.at[wid], src_v)
    pltpu.sync_copy(dst_h.at[wid], dst_v)

    def one_pass(h_h, out_h):
        def issue_gf(j, b):
            pltpu.async_copy(h_h.at[src_v.at[j]], rs_v.at[b], gf_sem[b])
            pltpu.async_copy(h_h.at[dst_v.at[j]], rd_v.at[b], gf_sem[b])

        def wait_gf(b):
            pltpu.make_async_copy(h_h.at[src_v.at[0]], rs_v.at[b], gf_sem[b]).wait()
            pltpu.make_async_copy(h_h.at[dst_v.at[0]], rd_v.at[b], gf_sem[b]).wait()

        def wait_w(b):
            pltpu.make_async_copy(rs_v.at[b], out_h.at[wid, 0], w_sem[b]).wait()

        for b in range(NBUF - 1):
            issue_gf(b, b)

        def round_body(r, carry):
            i0 = r * NBUF
            for b in range(NBUF):
                i = i0 + b
                bj = (b + NBUF - 1) % NBUF
                j = i + NBUF - 1

                @pl.when(j < N_CHUNKS)
                def _prefetch():
                    @pl.when(j >= NBUF)
                    def _drain():
                        wait_w(bj)

                    issue_gf(j, bj)

                wait_gf(b)
                _mul_rows(rs_v.at[b], rd_v.at[b])
                pltpu.async_copy(rs_v.at[b], out_h.at[wid, i], w_sem[b])
            return carry

        lax.fori_loop(0, N_ROUNDS, round_body, 0)
        for b in range(NBUF):
            wait_w(b)

    one_pass(hlg_h, olg_h)
    one_pass(hhg_h, ohg_h)
    one_pass(hll_h, oll_h)
    one_pass(hhl_h, ohl_h)


def _pair_products(hs_g, hs_l, src3, dst3):
    k = functools.partial(
        pl.kernel,
        out_type=[jax.ShapeDtypeStruct((NW, N_CHUNKS, CHUNK, HALF), jnp.float32)] * 4,
        mesh=_MESH,
        scratch_types=[
            pltpu.VMEM((N_CHUNKS, CHUNK), jnp.int32),
            pltpu.VMEM((N_CHUNKS, CHUNK), jnp.int32),
            pltpu.VMEM((NBUF, CHUNK, HALF), jnp.float32),
            pltpu.VMEM((NBUF, CHUNK, HALF), jnp.float32),
        ] + [pltpu.SemaphoreType.DMA] * (2 * NBUF),
        compiler_params=pltpu.CompilerParams(use_tc_tiling_on_sc=False),
    )(_pair_body)
    return k(hs_g[0], hs_g[1], hs_l[0], hs_l[1], src3, dst3)


def _final_body(hlg_ref, hhg_ref, hll_ref, hhl_ref, eag_ref, eal_ref,
                w1lg_ref, w1hg_ref, w1bg_ref, b1g_ref, w2g_ref, b2g_ref,
                w3g_ref, b3g_ref,
                w1ll_ref, w1hl_ref, w1bl_ref, b1l_ref, w2l_ref, b2l_ref,
                w3l_ref, b3l_ref,
                out_ref):
    def branch(hlo_ref, hhi_ref, ea_ref, w1l, w1h, w1b, b1, w2, b2, w3, b3):
        hlo = hlo_ref[...].reshape(EB, HALF)
        hhi = hhi_ref[...].reshape(EB, HALF)
        x = _silu(jnp.dot(hlo, w1l[...], preferred_element_type=jnp.float32)
                  + jnp.dot(hhi, w1h[...], preferred_element_type=jnp.float32)
                  + jnp.dot(ea_ref[...], w1b[...], preferred_element_type=jnp.float32)
                  + b1[...])
        x = _silu(jnp.dot(x, w2[...], preferred_element_type=jnp.float32) + b2[...])
        return jnp.dot(x, w3[...], preferred_element_type=jnp.float32) + b3[...]

    og = branch(hlg_ref, hhg_ref, eag_ref, w1lg_ref, w1hg_ref, w1bg_ref,
                b1g_ref, w2g_ref, b2g_ref, w3g_ref, b3g_ref)
    ol = branch(hll_ref, hhl_ref, eal_ref, w1ll_ref, w1hl_ref, w1bl_ref,
                b1l_ref, w2l_ref, b2l_ref, w3l_ref, b3l_ref)
    out_ref[...] = jnp.concatenate([og, ol], axis=1)


def _final_mlp(hh_g, hh_l, ea_g, ea_l,
               Wm1_g, bm1_g, Wm2_g, bm2_g, Wm3_g, bm3_g,
               Wm1_l, bm1_l, Wm2_l, bm2_l, Wm3_l, bm3_l):
    w_spec = lambda shape: pl.BlockSpec(shape, lambda w, g: (0, 0))
    h_spec = pl.BlockSpec((1, GB, CHUNK, HALF), lambda w, g: (w, g, 0, 0))
    e_spec = pl.BlockSpec((EB, HID_C), lambda w, g: (NG * w + g, 0))
    return pl.pallas_call(
        _final_body,
        grid=(NW, NG),
        in_specs=[
            h_spec, h_spec, h_spec, h_spec, e_spec, e_spec,
            w_spec((HALF, HID_C)), w_spec((HALF, HID_C)),
            w_spec((HID_C, HID_C)), w_spec((1, HID_C)),
            w_spec((HID_C, 64)), w_spec((1, 64)), w_spec((64, 1)), w_spec((1, 1)),
            w_spec((HALF, HID_C)), w_spec((HALF, HID_C)),
            w_spec((HID_C, HID_C)), w_spec((1, HID_C)),
            w_spec((HID_C, 64)), w_spec((1, 64)), w_spec((64, 1)), w_spec((1, 1)),
        ],
        out_specs=pl.BlockSpec((EB, 2), lambda w, g: (NG * w + g, 0)),
        out_shape=jax.ShapeDtypeStruct((N_EDGES_C, 2), jnp.float32),
    )(hh_g[0], hh_g[1], hh_l[0], hh_l[1], ea_g, ea_l,
      Wm1_g[:HALF], Wm1_g[HALF:HID_C], Wm1_g[HID_C:], bm1_g.reshape(1, HID_C),
      Wm2_g, bm2_g.reshape(1, 64), Wm3_g, bm3_g.reshape(1, 1),
      Wm1_l[:HALF], Wm1_l[HALF:HID_C], Wm1_l[HID_C:], bm1_l.reshape(1, HID_C),
      Wm2_l, bm2_l.reshape(1, 64), Wm3_l, bm3_l.reshape(1, 1))


def kernel(atom_type, r_feat, p_feat, pos, bond_index, bond_type, batch,
           atom_tab, W_feat,
           We1_g, be1_g, We2_g, be2_g, bond_g, Wconv_g, bconv_g,
           Wm1_g, bm1_g, Wm2_g, bm2_g, Wm3_g, bm3_g,
           We1_l, be1_l, We2_l, be2_l, bond_l, Wconv_l, bconv_l,
           Wm1_l, bm1_l, Wm2_l, bm2_l, Wm3_l, bm3_l):
    src = bond_index[0].astype(jnp.int32)
    dst = bond_index[1].astype(jnp.int32)
    src3 = src.reshape(NW, N_CHUNKS, CHUNK)
    dst3 = dst.reshape(NW, N_CHUNKS, CHUNK)
    zeros_block = jnp.zeros((RTA, HALF), jnp.float32)

    z = _node_embed(atom_type, r_feat, p_feat, atom_tab, W_feat)
    d2 = _edge_d2(pos, src, dst)
    ea_g, ea_l = _edge_attr(d2, bond_type, We1_g, be1_g, We2_g, be2_g, bond_g,
                            We1_l, be1_l, We2_l, be2_l, bond_l)

    hs_g = z
    hs_l = z
    for l in range(2):
        fg_lo, fg_hi, fl_lo, fl_hi = _filters(
            ea_g, ea_l,
            Wconv_g[l, 0], bconv_g[l, 0], Wconv_g[l, 1], bconv_g[l, 1],
            Wconv_l[l, 0], bconv_l[l, 0], Wconv_l[l, 1], bconv_l[l, 1])
        agg_g = _gather_mul_scatter(hs_g[0], hs_g[1], fg_lo, fg_hi,
                                    src3, dst3, zeros_block)
        agg_l = _gather_mul_scatter(hs_l[0], hs_l[1], fl_lo, fl_hi,
                                    src3, dst3, zeros_block)
        hs_g, hs_l = _node_update(hs_g, hs_l, agg_g, agg_l,
                                  Wconv_g[l, 2], bconv_g[l, 2],
                                  Wconv_l[l, 2], bconv_l[l, 2])

    hh = _pair_products(hs_g, hs_l, src3, dst3)
    return _final_mlp((hh[0], hh[1]), (hh[2], hh[3]), ea_g, ea_l,
                      Wm1_g, bm1_g, Wm2_g, bm2_g, Wm3_g, bm3_g,
                      Wm1_l, bm1_l, Wm2_l, bm2_l, Wm3_l, bm3_l)


# speedup vs baseline: 1.5822x; 1.5822x over previous
import functools

import jax
import jax.numpy as jnp
from jax import lax
from jax.experimental import pallas as pl
from jax.experimental.pallas import tpu as pltpu
from jax.experimental.pallas import tpu_sc as plsc

N_NODES_C = 10000
N_EDGES_C = 320000
HID_C = 128
HALF = 64
NUM_BOND_C = 24
NUM_ATOM_C = 100

NC = 2
NS = 16
NW = NC * NS
EPW = N_EDGES_C // NW
CHUNK = 80
N_CHUNKS = EPW // CHUNK
NBUF = 5
N_ROUNDS = N_CHUNKS // NBUF
RTA = 624
TAIL = N_NODES_C - NS * RTA

_MESH = plsc.VectorSubcoreMesh(core_axis_name="c", subcore_axis_name="s")


def _silu(x):
    return x * (1.0 / (1.0 + jnp.exp(-x)))


def _mul_rows(rows, other):
    def mulrow(r, cc):
        for j in range(HALF // 16):
            sl = pl.ds(j * 16, 16)
            rows[r, sl] = rows[r, sl] * other[r, sl]
        return cc

    lax.fori_loop(0, CHUNK, mulrow, 0)


def _node_embed_body(at_ref, rf_ref, pf_ref, tab_ref, wf_ref, zlo_ref, zhi_ref):
    oh = (at_ref[...] == lax.broadcasted_iota(jnp.int32, (at_ref.shape[0], NUM_ATOM_C), 1)
          ).astype(jnp.float32)
    emb = jnp.dot(oh, tab_ref[...], preferred_element_type=jnp.float32)
    fr = jnp.dot(rf_ref[...], wf_ref[...], preferred_element_type=jnp.float32)
    fp = jnp.dot(pf_ref[...], wf_ref[...], preferred_element_type=jnp.float32)
    zlo_ref[...] = emb + fr
    zhi_ref[...] = fp - fr


def _node_embed(atom_type, r_feat, p_feat, atom_tab, W_feat):
    nb = 1000
    grid = N_NODES_C // nb
    return pl.pallas_call(
        _node_embed_body,
        grid=(grid,),
        in_specs=[
            pl.BlockSpec((nb, 1), lambda i: (i, 0)),
            pl.BlockSpec((nb, 128), lambda i: (i, 0)),
            pl.BlockSpec((nb, 128), lambda i: (i, 0)),
            pl.BlockSpec((NUM_ATOM_C, HALF), lambda i: (0, 0)),
            pl.BlockSpec((128, HALF), lambda i: (0, 0)),
        ],
        out_specs=[pl.BlockSpec((nb, HALF), lambda i: (i, 0))] * 2,
        out_shape=[jax.ShapeDtypeStruct((N_NODES_C, HALF), jnp.float32)] * 2,
    )(atom_type.reshape(N_NODES_C, 1).astype(jnp.int32), r_feat, p_feat,
      atom_tab, W_feat)


def _d2_sc_body(px_h, py_h, pz_h, src_h, dst_h, d2_h,
                px_v, py_v, pz_v, src_v, dst_v, out_v):
    wid = lax.axis_index("s") * NC + lax.axis_index("c")
    base = wid * EPW
    pltpu.sync_copy(px_h, px_v)
    pltpu.sync_copy(py_h, py_v)
    pltpu.sync_copy(pz_h, pz_v)
    pltpu.sync_copy(src_h.at[pl.ds(base, EPW)], src_v)
    pltpu.sync_copy(dst_h.at[pl.ds(base, EPW)], dst_v)

    def body(i, carry):
        sl = pl.ds(i * 16, 16)
        s = src_v[sl]
        t = dst_v[sl]
        dx = plsc.load_gather(px_v, [t]) - plsc.load_gather(px_v, [s])
        dy = plsc.load_gather(py_v, [t]) - plsc.load_gather(py_v, [s])
        dz = plsc.load_gather(pz_v, [t]) - plsc.load_gather(pz_v, [s])
        out_v[sl] = dx * dx + dy * dy + dz * dz
        return carry

    lax.fori_loop(0, EPW // 16, body, 0)
    pltpu.sync_copy(out_v, d2_h.at[pl.ds(base, EPW)])


def _edge_d2(pos, src, dst):
    k = functools.partial(
        pl.kernel,
        out_type=jax.ShapeDtypeStruct((N_EDGES_C,), jnp.float32),
        mesh=_MESH,
        scratch_types=[
            pltpu.VMEM((N_NODES_C,), jnp.float32),
            pltpu.VMEM((N_NODES_C,), jnp.float32),
            pltpu.VMEM((N_NODES_C,), jnp.float32),
            pltpu.VMEM((EPW,), jnp.int32),
            pltpu.VMEM((EPW,), jnp.int32),
            pltpu.VMEM((EPW,), jnp.float32),
        ],
        compiler_params=pltpu.CompilerParams(needs_layout_passes=False),
    )(_d2_sc_body)
    pos_t = pos.T
    return k(pos_t[0], pos_t[1], pos_t[2], src, dst)


def _edge_attr_body(d2_ref, bt_ref,
                    we1g_ref, be1g_ref, we2g_ref, be2g_ref, bondg_ref,
                    we1l_ref, be1l_ref, we2l_ref, be2l_ref, bondl_ref,
                    eag_ref, eal_ref):
    d = jnp.sqrt(d2_ref[...])
    oh = (bt_ref[...] == lax.broadcasted_iota(jnp.int32, (bt_ref.shape[0], NUM_BOND_C), 1)
          ).astype(jnp.float32)

    def branch(we1, be1, we2, be2, bond, out_ref):
        e1 = _silu(d * we1[...] + be1[...])
        e = jnp.dot(e1, we2[...], preferred_element_type=jnp.float32) + be2[...]
        bemb = jnp.dot(oh, bond[...], preferred_element_type=jnp.float32)
        out_ref[...] = e * bemb

    branch(we1g_ref, be1g_ref, we2g_ref, be2g_ref, bondg_ref, eag_ref)
    branch(we1l_ref, be1l_ref, we2l_ref, be2l_ref, bondl_ref, eal_ref)


def _edge_attr(d2, bond_type, We1_g, be1_g, We2_g, be2_g, bond_g,
               We1_l, be1_l, We2_l, be2_l, bond_l):
    eb = 2560
    grid = N_EDGES_C // eb
    w_spec = lambda shape: pl.BlockSpec(shape, lambda i: (0, 0))
    return pl.pallas_call(
        _edge_attr_body,
        grid=(grid,),
        in_specs=[
            pl.BlockSpec((eb, 1), lambda i: (i, 0)),
            pl.BlockSpec((eb, 1), lambda i: (i, 0)),
            w_spec((1, HID_C)), w_spec((1, HID_C)), w_spec((HID_C, HID_C)),
            w_spec((1, HID_C)), w_spec((NUM_BOND_C, HID_C)),
            w_spec((1, HID_C)), w_spec((1, HID_C)), w_spec((HID_C, HID_C)),
            w_spec((1, HID_C)), w_spec((NUM_BOND_C, HID_C)),
        ],
        out_specs=[pl.BlockSpec((eb, HID_C), lambda i: (i, 0))] * 2,
        out_shape=[jax.ShapeDtypeStruct((N_EDGES_C, HID_C), jnp.float32)] * 2,
    )(d2.reshape(N_EDGES_C, 1), bond_type.reshape(N_EDGES_C, 1).astype(jnp.int32),
      We1_g, be1_g.reshape(1, HID_C), We2_g, be2_g.reshape(1, HID_C), bond_g,
      We1_l, be1_l.reshape(1, HID_C), We2_l, be2_l.reshape(1, HID_C), bond_l)


GB = 25
EB = GB * CHUNK
NG = N_CHUNKS // GB


def _filters_body(eag_ref, eal_ref,
                  wf1g_ref, bf1g_ref, wf2g_ref, bf2g_ref,
                  wf1l_ref, bf1l_ref, wf2l_ref, bf2l_ref,
                  fg_ref, fl_ref):
    def branch(ea_ref, wf1, bf1, wf2, bf2, out_ref):
        t = _silu(jnp.dot(ea_ref[...], wf1[...], preferred_element_type=jnp.float32)
                  + bf1[...])
        f = jnp.dot(t, wf2[...], preferred_element_type=jnp.float32) + bf2[...]
        out_ref[...] = f.reshape(1, GB, CHUNK, HID_C)

    branch(eag_ref, wf1g_ref, bf1g_ref, wf2g_ref, bf2g_ref, fg_ref)
    branch(eal_ref, wf1l_ref, bf1l_ref, wf2l_ref, bf2l_ref, fl_ref)


def _filters(ea_g, ea_l, wf1_g, bf1_g, wf2_g, bf2_g, wf1_l, bf1_l, wf2_l, bf2_l):
    w_spec = lambda shape: pl.BlockSpec(shape, lambda w, g: (0, 0))
    e_spec = pl.BlockSpec((EB, HID_C), lambda w, g: (NG * w + g, 0))
    f_spec = pl.BlockSpec((1, GB, CHUNK, HID_C), lambda w, g: (w, g, 0, 0))
    return pl.pallas_call(
        _filters_body,
        grid=(NW, NG),
        in_specs=[
            e_spec, e_spec,
            w_spec((HID_C, HID_C)), w_spec((1, HID_C)),
            w_spec((HID_C, HID_C)), w_spec((1, HID_C)),
            w_spec((HID_C, HID_C)), w_spec((1, HID_C)),
            w_spec((HID_C, HID_C)), w_spec((1, HID_C)),
        ],
        out_specs=[f_spec] * 2,
        out_shape=[jax.ShapeDtypeStruct((NW, N_CHUNKS, CHUNK, HID_C),
                                        jnp.float32)] * 2,
    )(ea_g, ea_l,
      wf1_g, bf1_g.reshape(1, HID_C), wf2_g, bf2_g.reshape(1, HID_C),
      wf1_l, bf1_l.reshape(1, HID_C), wf2_l, bf2_l.reshape(1, HID_C))


def _gms_body(hlo_h, hhi_h, f_h, src_h, dst_h, zeros_h, out_h,
              src_v, dst_v, rows_v, filt_v, agg_sh, *sems):
    gf_sem = sems[:NBUF]
    s_sem = sems[NBUF:]
    c = lax.axis_index("c")
    s = lax.axis_index("s")
    wid = s * NC + c

    pltpu.sync_copy(src_h.at[wid], src_v)
    pltpu.sync_copy(dst_h.at[wid], dst_v)

    for half, h_h in enumerate((hlo_h, hhi_h)):
        fsl = pl.ds(half * HALF, HALF)

        def issue_gf(j, b):
            pltpu.async_copy(h_h.at[src_v.at[j]], rows_v.at[b], gf_sem[b])
            pltpu.async_copy(f_h.at[wid, j, :, fsl], filt_v.at[b], gf_sem[b])

        def wait_gf(b):
            pltpu.make_async_copy(h_h.at[src_v.at[0]], rows_v.at[b],
                                  gf_sem[b]).wait()
            pltpu.make_async_copy(f_h.at[wid, 0, :, fsl], filt_v.at[b],
                                  gf_sem[b]).wait()

        def wait_scatter(b):
            pltpu.make_async_copy(rows_v.at[b], agg_sh.at[dst_v.at[0]],
                                  s_sem[b]).wait()

        for b in range(NBUF - 1):
            issue_gf(b, b)

        pltpu.sync_copy(zeros_h, agg_sh.at[pl.ds(s * RTA, RTA)])

        @pl.when(s == 0)
        def _zero_tail():
            pltpu.sync_copy(zeros_h.at[pl.ds(0, TAIL)],
                            agg_sh.at[pl.ds(NS * RTA, TAIL)])

        plsc.subcore_barrier()

        def round_body(r, carry):
            i0 = r * NBUF
            for b in range(NBUF):
                i = i0 + b
                bj = (b + NBUF - 1) % NBUF
                j = i + NBUF - 1

                @pl.when(j < N_CHUNKS)
                def _prefetch():
                    @pl.when(j >= NBUF)
                    def _drain():
                        wait_scatter(bj)

                    issue_gf(j, bj)

                wait_gf(b)
                _mul_rows(rows_v.at[b], filt_v.at[b])
                pltpu.async_copy(rows_v.at[b], agg_sh.at[dst_v.at[i]], s_sem[b],
                                 add=True)
            return carry

        lax.fori_loop(0, N_ROUNDS, round_body, 0)
        for b in range(NBUF):
            wait_scatter(b)
        plsc.subcore_barrier()
        pltpu.sync_copy(agg_sh.at[pl.ds(s * RTA, RTA)],
                        out_h.at[c, half, pl.ds(s * RTA, RTA)])

        @pl.when(s == 0)
        def _copy_tail():
            pltpu.sync_copy(agg_sh.at[pl.ds(NS * RTA, TAIL)],
                            out_h.at[c, half, pl.ds(NS * RTA, TAIL)])

        plsc.subcore_barrier()


def _gather_mul_scatter(h_lo, h_hi, f4, src3, dst3, zeros_block):
    k = functools.partial(
        pl.kernel,
        out_type=jax.ShapeDtypeStruct((NC, 2, N_NODES_C, HALF), jnp.float32),
        mesh=_MESH,
        scratch_types=[
            pltpu.VMEM((N_CHUNKS, CHUNK), jnp.int32),
            pltpu.VMEM((N_CHUNKS, CHUNK), jnp.int32),
            pltpu.VMEM((NBUF, CHUNK, HALF), jnp.float32),
            pltpu.VMEM((NBUF, CHUNK, HALF), jnp.float32),
            pltpu.VMEM_SHARED((N_NODES_C, HALF), jnp.float32),
        ] + [pltpu.SemaphoreType.DMA] * (2 * NBUF),
        compiler_params=pltpu.CompilerParams(use_tc_tiling_on_sc=False),
    )(_gms_body)
    return k(h_lo, h_hi, f4, src3, dst3, zeros_block)


def _update_body(hlg_ref, hhg_ref, hll_ref, hhl_ref, ag_ref, al_ref,
                 wug_ref, bug_ref, wul_ref, bul_ref,
                 olg_ref, ohg_ref, oll_ref, ohl_ref):
    def branch(hlo_ref, hhi_ref, a_ref, wu, bu, olo_ref, ohi_ref):
        agg = jnp.concatenate([a_ref[0, 0] + a_ref[1, 0],
                               a_ref[0, 1] + a_ref[1, 1]], axis=1)
        x = _silu(jnp.dot(agg, wu[...], preferred_element_type=jnp.float32)
                  + bu[...])
        olo_ref[...] = hlo_ref[...] + x[:, :HALF]
        ohi_ref[...] = hhi_ref[...] + x[:, HALF:]

    branch(hlg_ref, hhg_ref, ag_ref, wug_ref, bug_ref, olg_ref, ohg_ref)
    branch(hll_ref, hhl_ref, al_ref, wul_ref, bul_ref, oll_ref, ohl_ref)


def _node_update(hs_g, hs_l, agg_g, agg_l, wu_g, bu_g, wu_l, bu_l):
    nb = 1000
    grid = N_NODES_C // nb
    w_spec = lambda shape: pl.BlockSpec(shape, lambda i: tuple([0] * len(shape)))
    h_spec = pl.BlockSpec((nb, HALF), lambda i: (i, 0))
    a_spec = pl.BlockSpec((NC, 2, nb, HALF), lambda i: (0, 0, i, 0))
    out = pl.pallas_call(
        _update_body,
        grid=(grid,),
        in_specs=[h_spec, h_spec, h_spec, h_spec, a_spec, a_spec,
                  w_spec((HID_C, HID_C)), w_spec((1, HID_C)),
                  w_spec((HID_C, HID_C)), w_spec((1, HID_C))],
        out_specs=[h_spec] * 4,
        out_shape=[jax.ShapeDtypeStruct((N_NODES_C, HALF), jnp.float32)] * 4,
    )(hs_g[0], hs_g[1], hs_l[0], hs_l[1], agg_g, agg_l,
      wu_g, bu_g.reshape(1, HID_C), wu_l, bu_l.reshape(1, HID_C))
    return (out[0], out[1]), (out[2], out[3])


def _pair_body(hlg_h, hhg_h, hll_h, hhl_h, src_h, dst_h, og_h, ol_h,
               src_v, dst_v, rs_v, rd_v, *sems):
    gf_sem = sems[:NBUF]
    w_sem = sems[NBUF:]
    wid = lax.axis_index("s") * NC + lax.axis_index("c")

    pltpu.sync_copy(src_h.at[wid], src_v)
    pltpu.sync_copy(dst_h.at[wid], dst_v)

    def one_pass(h_h, out_h, half):
        fsl = pl.ds(half * HALF, HALF)

        def issue_gf(j, b):
            pltpu.async_copy(h_h.at[src_v.at[j]], rs_v.at[b], gf_sem[b])
            pltpu.async_copy(h_h.at[dst_v.at[j]], rd_v.at[b], gf_sem[b])

        def wait_gf(b):
            pltpu.make_async_copy(h_h.at[src_v.at[0]], rs_v.at[b], gf_sem[b]).wait()
            pltpu.make_async_copy(h_h.at[dst_v.at[0]], rd_v.at[b], gf_sem[b]).wait()

        def wait_w(b):
            pltpu.make_async_copy(rs_v.at[b], out_h.at[wid, 0, :, fsl],
                                  w_sem[b]).wait()

        for b in range(NBUF - 1):
            issue_gf(b, b)

        def round_body(r, carry):
            i0 = r * NBUF
            for b in range(NBUF):
                i = i0 + b
                bj = (b + NBUF - 1) % NBUF
                j = i + NBUF - 1

                @pl.when(j < N_CHUNKS)
                def _prefetch():
                    @pl.when(j >= NBUF)
                    def _drain():
                        wait_w(bj)

                    issue_gf(j, bj)

                wait_gf(b)
                _mul_rows(rs_v.at[b], rd_v.at[b])
                pltpu.async_copy(rs_v.at[b], out_h.at[wid, i, :, fsl], w_sem[b])
            return carry

        lax.fori_loop(0, N_ROUNDS, round_body, 0)
        for b in range(NBUF):
            wait_w(b)

    one_pass(hlg_h, og_h, 0)
    one_pass(hhg_h, og_h, 1)
    one_pass(hll_h, ol_h, 0)
    one_pass(hhl_h, ol_h, 1)


def _pair_products(hs_g, hs_l, src3, dst3):
    k = functools.partial(
        pl.kernel,
        out_type=[jax.ShapeDtypeStruct((NW, N_CHUNKS, CHUNK, HID_C),
                                       jnp.float32)] * 2,
        mesh=_MESH,
        scratch_types=[
            pltpu.VMEM((N_CHUNKS, CHUNK), jnp.int32),
            pltpu.VMEM((N_CHUNKS, CHUNK), jnp.int32),
            pltpu.VMEM((NBUF, CHUNK, HALF), jnp.float32),
            pltpu.VMEM((NBUF, CHUNK, HALF), jnp.float32),
        ] + [pltpu.SemaphoreType.DMA] * (2 * NBUF),
        compiler_params=pltpu.CompilerParams(use_tc_tiling_on_sc=False),
    )(_pair_body)
    return k(hs_g[0], hs_g[1], hs_l[0], hs_l[1], src3, dst3)


def _final_body(hhg_ref, hhl_ref, eag_ref, eal_ref,
                w1ag_ref, w1bg_ref, b1g_ref, w2g_ref, b2g_ref, w3g_ref, b3g_ref,
                w1al_ref, w1bl_ref, b1l_ref, w2l_ref, b2l_ref, w3l_ref, b3l_ref,
                out_ref):
    def branch(hh_ref, ea_ref, w1a, w1b, b1, w2, b2, w3, b3):
        hh = hh_ref[...].reshape(EB, HID_C)
        x = _silu(jnp.dot(hh, w1a[...], preferred_element_type=jnp.float32)
                  + jnp.dot(ea_ref[...], w1b[...], preferred_element_type=jnp.float32)
                  + b1[...])
        x = _silu(jnp.dot(x, w2[...], preferred_element_type=jnp.float32) + b2[...])
        return jnp.dot(x, w3[...], preferred_element_type=jnp.float32) + b3[...]

    og = branch(hhg_ref, eag_ref, w1ag_ref, w1bg_ref, b1g_ref, w2g_ref,
                b2g_ref, w3g_ref, b3g_ref)
    ol = branch(hhl_ref, eal_ref, w1al_ref, w1bl_ref, b1l_ref, w2l_ref,
                b2l_ref, w3l_ref, b3l_ref)
    out_ref[...] = jnp.concatenate([og, ol], axis=1)


def _final_mlp(hh_g, hh_l, ea_g, ea_l,
               Wm1_g, bm1_g, Wm2_g, bm2_g, Wm3_g, bm3_g,
               Wm1_l, bm1_l, Wm2_l, bm2_l, Wm3_l, bm3_l):
    w_spec = lambda shape: pl.BlockSpec(shape, lambda w, g: (0, 0))
    h_spec = pl.BlockSpec((1, GB, CHUNK, HID_C), lambda w, g: (w, g, 0, 0))
    e_spec = pl.BlockSpec((EB, HID_C), lambda w, g: (NG * w + g, 0))
    return pl.pallas_call(
        _final_body,
        grid=(NW, NG),
        in_specs=[
            h_spec, h_spec, e_spec, e_spec,
            w_spec((HID_C, HID_C)), w_spec((HID_C, HID_C)), w_spec((1, HID_C)),
            w_spec((HID_C, 64)), w_spec((1, 64)), w_spec((64, 1)), w_spec((1, 1)),
            w_spec((HID_C, HID_C)), w_spec((HID_C, HID_C)), w_spec((1, HID_C)),
            w_spec((HID_C, 64)), w_spec((1, 64)), w_spec((64, 1)), w_spec((1, 1)),
        ],
        out_specs=pl.BlockSpec((EB, 2), lambda w, g: (NG * w + g, 0)),
        out_shape=jax.ShapeDtypeStruct((N_EDGES_C, 2), jnp.float32),
    )(hh_g, hh_l, ea_g, ea_l,
      Wm1_g[:HID_C], Wm1_g[HID_C:], bm1_g.reshape(1, HID_C),
      Wm2_g, bm2_g.reshape(1, 64), Wm3_g, bm3_g.reshape(1, 1),
      Wm1_l[:HID_C], Wm1_l[HID_C:], bm1_l.reshape(1, HID_C),
      Wm2_l, bm2_l.reshape(1, 64), Wm3_l, bm3_l.reshape(1, 1))


def kernel(atom_type, r_feat, p_feat, pos, bond_index, bond_type, batch,
           atom_tab, W_feat,
           We1_g, be1_g, We2_g, be2_g, bond_g, Wconv_g, bconv_g,
           Wm1_g, bm1_g, Wm2_g, bm2_g, Wm3_g, bm3_g,
           We1_l, be1_l, We2_l, be2_l, bond_l, Wconv_l, bconv_l,
           Wm1_l, bm1_l, Wm2_l, bm2_l, Wm3_l, bm3_l):
    src = bond_index[0].astype(jnp.int32)
    dst = bond_index[1].astype(jnp.int32)
    src3 = src.reshape(NW, N_CHUNKS, CHUNK)
    dst3 = dst.reshape(NW, N_CHUNKS, CHUNK)
    zeros_block = jnp.zeros((RTA, HALF), jnp.float32)

    z = _node_embed(atom_type, r_feat, p_feat, atom_tab, W_feat)
    d2 = _edge_d2(pos, src, dst)
    ea_g, ea_l = _edge_attr(d2, bond_type, We1_g, be1_g, We2_g, be2_g, bond_g,
                            We1_l, be1_l, We2_l, be2_l, bond_l)

    hs_g = z
    hs_l = z
    for l in range(2):
        f_g, f_l = _filters(
            ea_g, ea_l,
            Wconv_g[l, 0], bconv_g[l, 0], Wconv_g[l, 1], bconv_g[l, 1],
            Wconv_l[l, 0], bconv_l[l, 0], Wconv_l[l, 1], bconv_l[l, 1])
        agg_g = _gather_mul_scatter(hs_g[0], hs_g[1], f_g, src3, dst3,
                                    zeros_block)
        agg_l = _gather_mul_scatter(hs_l[0], hs_l[1], f_l, src3, dst3,
                                    zeros_block)
        hs_g, hs_l = _node_update(hs_g, hs_l, agg_g, agg_l,
                                  Wconv_g[l, 2], bconv_g[l, 2],
                                  Wconv_l[l, 2], bconv_l[l, 2])

    hh_g, hh_l = _pair_products(hs_g, hs_l, src3, dst3)
    return _final_mlp(hh_g, hh_l, ea_g, ea_l,
                      Wm1_g, bm1_g, Wm2_g, bm2_g, Wm3_g, bm3_g,
                      Wm1_l, bm1_l, Wm2_l, bm2_l, Wm3_l, bm3_l)


# speedup vs baseline: 1.6515x; 1.0438x over previous
import functools

import jax
import jax.numpy as jnp
from jax import lax
from jax.experimental import pallas as pl
from jax.experimental.pallas import tpu as pltpu
from jax.experimental.pallas import tpu_sc as plsc

N_NODES_C = 10000
N_EDGES_C = 320000
HID_C = 128
HALF = 64
NUM_BOND_C = 24
NUM_ATOM_C = 100

NC = 2
NS = 16
NW = NC * NS
EPW = N_EDGES_C // NW
CHUNK = 80
N_CHUNKS = EPW // CHUNK
NBUF = 5
N_ROUNDS = N_CHUNKS // NBUF
RTA = 624
TAIL = N_NODES_C - NS * RTA

_MESH = plsc.VectorSubcoreMesh(core_axis_name="c", subcore_axis_name="s")


def _silu(x):
    return x * (1.0 / (1.0 + jnp.exp(-x)))


def _mul_rows(rows, other):
    def mulrow(r, cc):
        for j in range(HALF // 16):
            sl = pl.ds(j * 16, 16)
            rows[r, sl] = rows[r, sl] * other[r, sl]
        return cc

    lax.fori_loop(0, CHUNK, mulrow, 0)


def _node_embed_body(at_ref, rf_ref, pf_ref, tab_ref, wf_ref, zlo_ref, zhi_ref):
    oh = (at_ref[...] == lax.broadcasted_iota(jnp.int32, (at_ref.shape[0], NUM_ATOM_C), 1)
          ).astype(jnp.float32)
    emb = jnp.dot(oh, tab_ref[...], preferred_element_type=jnp.float32)
    fr = jnp.dot(rf_ref[...], wf_ref[...], preferred_element_type=jnp.float32)
    fp = jnp.dot(pf_ref[...], wf_ref[...], preferred_element_type=jnp.float32)
    zlo_ref[...] = emb + fr
    zhi_ref[...] = fp - fr


def _node_embed(atom_type, r_feat, p_feat, atom_tab, W_feat):
    nb = 1000
    grid = N_NODES_C // nb
    return pl.pallas_call(
        _node_embed_body,
        grid=(grid,),
        in_specs=[
            pl.BlockSpec((nb, 1), lambda i: (i, 0)),
            pl.BlockSpec((nb, 128), lambda i: (i, 0)),
            pl.BlockSpec((nb, 128), lambda i: (i, 0)),
            pl.BlockSpec((NUM_ATOM_C, HALF), lambda i: (0, 0)),
            pl.BlockSpec((128, HALF), lambda i: (0, 0)),
        ],
        out_specs=[pl.BlockSpec((nb, HALF), lambda i: (i, 0))] * 2,
        out_shape=[jax.ShapeDtypeStruct((N_NODES_C, HALF), jnp.float32)] * 2,
    )(atom_type.reshape(N_NODES_C, 1).astype(jnp.int32), r_feat, p_feat,
      atom_tab, W_feat)


def _d2_sc_body(px_h, py_h, pz_h, src_h, dst_h, d2_h,
                px_v, py_v, pz_v, src_v, dst_v, out_v):
    wid = lax.axis_index("s") * NC + lax.axis_index("c")
    base = wid * EPW
    pltpu.sync_copy(px_h, px_v)
    pltpu.sync_copy(py_h, py_v)
    pltpu.sync_copy(pz_h, pz_v)
    pltpu.sync_copy(src_h.at[pl.ds(base, EPW)], src_v)
    pltpu.sync_copy(dst_h.at[pl.ds(base, EPW)], dst_v)

    def body(i, carry):
        sl = pl.ds(i * 16, 16)
        s = src_v[sl]
        t = dst_v[sl]
        dx = plsc.load_gather(px_v, [t]) - plsc.load_gather(px_v, [s])
        dy = plsc.load_gather(py_v, [t]) - plsc.load_gather(py_v, [s])
        dz = plsc.load_gather(pz_v, [t]) - plsc.load_gather(pz_v, [s])
        out_v[sl] = dx * dx + dy * dy + dz * dz
        return carry

    lax.fori_loop(0, EPW // 16, body, 0)
    pltpu.sync_copy(out_v, d2_h.at[pl.ds(base, EPW)])


def _edge_d2(pos, src, dst):
    k = functools.partial(
        pl.kernel,
        out_type=jax.ShapeDtypeStruct((N_EDGES_C,), jnp.float32),
        mesh=_MESH,
        scratch_types=[
            pltpu.VMEM((N_NODES_C,), jnp.float32),
            pltpu.VMEM((N_NODES_C,), jnp.float32),
            pltpu.VMEM((N_NODES_C,), jnp.float32),
            pltpu.VMEM((EPW,), jnp.int32),
            pltpu.VMEM((EPW,), jnp.int32),
            pltpu.VMEM((EPW,), jnp.float32),
        ],
        compiler_params=pltpu.CompilerParams(needs_layout_passes=False),
    )(_d2_sc_body)
    pos_t = pos.T
    return k(pos_t[0], pos_t[1], pos_t[2], src, dst)


def _edge_attr_body(d2_ref, bt_ref,
                    we1g_ref, be1g_ref, we2g_ref, be2g_ref, bondg_ref,
                    we1l_ref, be1l_ref, we2l_ref, be2l_ref, bondl_ref,
                    eag_ref, eal_ref):
    eb = eag_ref.shape[0]
    d = jnp.sqrt(d2_ref[...])
    oh = (bt_ref[...] == lax.broadcasted_iota(jnp.int32, (eb, NUM_BOND_C), 1)
          ).astype(jnp.float32)

    def branch(we1, be1, we2, be2, bond, out_ref):
        e1 = _silu(d * we1[...] + be1[...])
        e = jnp.dot(e1, we2[...], preferred_element_type=jnp.float32) + be2[...]
        bemb = jnp.dot(oh, bond[...], preferred_element_type=jnp.float32)
        out_ref[...] = e * bemb

    branch(we1g_ref, be1g_ref, we2g_ref, be2g_ref, bondg_ref, eag_ref)
    branch(we1l_ref, be1l_ref, we2l_ref, be2l_ref, bondl_ref, eal_ref)


def _edge_attr(d2, bond_type, We1_g, be1_g, We2_g, be2_g, bond_g,
               We1_l, be1_l, We2_l, be2_l, bond_l):
    eb = 2560
    grid = N_EDGES_C // eb
    w_spec = lambda shape: pl.BlockSpec(shape, lambda i: (0, 0))
    return pl.pallas_call(
        _edge_attr_body,
        grid=(grid,),
        in_specs=[
            pl.BlockSpec((eb, 1), lambda i: (i, 0)),
            pl.BlockSpec((eb, 1), lambda i: (i, 0)),
            w_spec((1, HID_C)), w_spec((1, HID_C)), w_spec((HID_C, HID_C)),
            w_spec((1, HID_C)), w_spec((NUM_BOND_C, HID_C)),
            w_spec((1, HID_C)), w_spec((1, HID_C)), w_spec((HID_C, HID_C)),
            w_spec((1, HID_C)), w_spec((NUM_BOND_C, HID_C)),
        ],
        out_specs=[pl.BlockSpec((eb, HID_C), lambda i: (i, 0))] * 2,
        out_shape=[jax.ShapeDtypeStruct((N_EDGES_C, HID_C), jnp.float32)] * 2,
    )(d2.reshape(N_EDGES_C, 1),
      bond_type.astype(jnp.int32).reshape(N_EDGES_C, 1),
      We1_g, be1_g.reshape(1, HID_C), We2_g, be2_g.reshape(1, HID_C), bond_g,
      We1_l, be1_l.reshape(1, HID_C), We2_l, be2_l.reshape(1, HID_C), bond_l)


GB = 25
EB = GB * CHUNK
NG = N_CHUNKS // GB


def _filters_body(ea_ref,
                  wf1a_ref, bf1a_ref, wf2a_ref, bf2a_ref,
                  wf1b_ref, bf1b_ref, wf2b_ref, bf2b_ref,
                  fa_ref, fb_ref):
    ea = ea_ref[...]

    def layer(wf1, bf1, wf2, bf2, out_ref):
        t = _silu(jnp.dot(ea, wf1[...], preferred_element_type=jnp.float32)
                  + bf1[...])
        f = jnp.dot(t, wf2[...], preferred_element_type=jnp.float32) + bf2[...]
        out_ref[...] = f.reshape(1, GB, CHUNK, HID_C)

    layer(wf1a_ref, bf1a_ref, wf2a_ref, bf2a_ref, fa_ref)
    layer(wf1b_ref, bf1b_ref, wf2b_ref, bf2b_ref, fb_ref)


def _filters_branch(ea, Wconv, bconv):
    w_spec = lambda shape: pl.BlockSpec(shape, lambda w, g: (0, 0))
    e_spec = pl.BlockSpec((EB, HID_C), lambda w, g: (NG * w + g, 0))
    f_spec = pl.BlockSpec((1, GB, CHUNK, HID_C), lambda w, g: (w, g, 0, 0))
    return pl.pallas_call(
        _filters_body,
        grid=(NW, NG),
        in_specs=[
            e_spec,
            w_spec((HID_C, HID_C)), w_spec((1, HID_C)),
            w_spec((HID_C, HID_C)), w_spec((1, HID_C)),
            w_spec((HID_C, HID_C)), w_spec((1, HID_C)),
            w_spec((HID_C, HID_C)), w_spec((1, HID_C)),
        ],
        out_specs=[f_spec] * 2,
        out_shape=[jax.ShapeDtypeStruct((NW, N_CHUNKS, CHUNK, HID_C),
                                        jnp.float32)] * 2,
    )(ea,
      Wconv[0, 0], bconv[0, 0].reshape(1, HID_C),
      Wconv[0, 1], bconv[0, 1].reshape(1, HID_C),
      Wconv[1, 0], bconv[1, 0].reshape(1, HID_C),
      Wconv[1, 1], bconv[1, 1].reshape(1, HID_C))


def _gms_body(hlo_h, hhi_h, f_h, src_h, dst_h, zeros_h, out_h,
              src_v, dst_v, rows_v, filt_v, agg_sh, *sems):
    gf_sem = sems[:NBUF]
    s_sem = sems[NBUF:]
    c = lax.axis_index("c")
    s = lax.axis_index("s")
    wid = s * NC + c

    pltpu.sync_copy(src_h.at[wid], src_v)
    pltpu.sync_copy(dst_h.at[wid], dst_v)

    for half, h_h in enumerate((hlo_h, hhi_h)):
        fsl = pl.ds(half * HALF, HALF)

        def issue_gf(j, b):
            pltpu.async_copy(h_h.at[src_v.at[j]], rows_v.at[b], gf_sem[b])
            pltpu.async_copy(f_h.at[wid, j, :, fsl], filt_v.at[b], gf_sem[b])

        def wait_gf(b):
            pltpu.make_async_copy(h_h.at[src_v.at[0]], rows_v.at[b],
                                  gf_sem[b]).wait()
            pltpu.make_async_copy(f_h.at[wid, 0, :, fsl], filt_v.at[b],
                                  gf_sem[b]).wait()

        def wait_scatter(b):
            pltpu.make_async_copy(rows_v.at[b], agg_sh.at[dst_v.at[0]],
                                  s_sem[b]).wait()

        for b in range(NBUF - 1):
            issue_gf(b, b)

        pltpu.sync_copy(zeros_h, agg_sh.at[pl.ds(s * RTA, RTA)])

        @pl.when(s == 0)
        def _zero_tail():
            pltpu.sync_copy(zeros_h.at[pl.ds(0, TAIL)],
                            agg_sh.at[pl.ds(NS * RTA, TAIL)])

        plsc.subcore_barrier()

        def round_body(r, carry):
            i0 = r * NBUF
            for b in range(NBUF):
                i = i0 + b
                bj = (b + NBUF - 1) % NBUF
                j = i + NBUF - 1

                @pl.when(j < N_CHUNKS)
                def _prefetch():
                    @pl.when(j >= NBUF)
                    def _drain():
                        wait_scatter(bj)

                    issue_gf(j, bj)

                wait_gf(b)
                _mul_rows(rows_v.at[b], filt_v.at[b])
                pltpu.async_copy(rows_v.at[b], agg_sh.at[dst_v.at[i]], s_sem[b],
                                 add=True)
            return carry

        lax.fori_loop(0, N_ROUNDS, round_body, 0)
        for b in range(NBUF):
            wait_scatter(b)
        plsc.subcore_barrier()
        pltpu.sync_copy(agg_sh.at[pl.ds(s * RTA, RTA)],
                        out_h.at[c, half, pl.ds(s * RTA, RTA)])

        @pl.when(s == 0)
        def _copy_tail():
            pltpu.sync_copy(agg_sh.at[pl.ds(NS * RTA, TAIL)],
                            out_h.at[c, half, pl.ds(NS * RTA, TAIL)])

        plsc.subcore_barrier()


def _gather_mul_scatter(h_lo, h_hi, f4, src3, dst3, zeros_block):
    k = functools.partial(
        pl.kernel,
        out_type=jax.ShapeDtypeStruct((NC, 2, N_NODES_C, HALF), jnp.float32),
        mesh=_MESH,
        scratch_types=[
            pltpu.VMEM((N_CHUNKS, CHUNK), jnp.int32),
            pltpu.VMEM((N_CHUNKS, CHUNK), jnp.int32),
            pltpu.VMEM((NBUF, CHUNK, HALF), jnp.float32),
            pltpu.VMEM((NBUF, CHUNK, HALF), jnp.float32),
            pltpu.VMEM_SHARED((N_NODES_C, HALF), jnp.float32),
        ] + [pltpu.SemaphoreType.DMA] * (2 * NBUF),
        compiler_params=pltpu.CompilerParams(use_tc_tiling_on_sc=False),
    )(_gms_body)
    return k(h_lo, h_hi, f4, src3, dst3, zeros_block)


def _update_body(hlo_ref, hhi_ref, a_ref, wu_ref, bu_ref, olo_ref, ohi_ref):
    agg = jnp.concatenate([a_ref[0, 0] + a_ref[1, 0],
                           a_ref[0, 1] + a_ref[1, 1]], axis=1)
    x = _silu(jnp.dot(agg, wu_ref[...], preferred_element_type=jnp.float32)
              + bu_ref[...])
    olo_ref[...] = hlo_ref[...] + x[:, :HALF]
    ohi_ref[...] = hhi_ref[...] + x[:, HALF:]


def _node_update(hs, agg, wu, bu):
    nb = 1000
    grid = N_NODES_C // nb
    w_spec = lambda shape: pl.BlockSpec(shape, lambda i: tuple([0] * len(shape)))
    h_spec = pl.BlockSpec((nb, HALF), lambda i: (i, 0))
    a_spec = pl.BlockSpec((NC, 2, nb, HALF), lambda i: (0, 0, i, 0))
    out = pl.pallas_call(
        _update_body,
        grid=(grid,),
        in_specs=[h_spec, h_spec, a_spec,
                  w_spec((HID_C, HID_C)), w_spec((1, HID_C))],
        out_specs=[h_spec] * 2,
        out_shape=[jax.ShapeDtypeStruct((N_NODES_C, HALF), jnp.float32)] * 2,
    )(hs[0], hs[1], agg, wu, bu.reshape(1, HID_C))
    return (out[0], out[1])


def _pair_body(hlo_h, hhi_h, src_h, dst_h, out4_h,
               src_v, dst_v, rs_v, rd_v, *sems):
    gf_sem = sems[:NBUF]
    w_sem = sems[NBUF:]
    wid = lax.axis_index("s") * NC + lax.axis_index("c")

    pltpu.sync_copy(src_h.at[wid], src_v)
    pltpu.sync_copy(dst_h.at[wid], dst_v)

    def one_pass(h_h, out_h, half):
        fsl = pl.ds(half * HALF, HALF)

        def issue_gf(j, b):
            pltpu.async_copy(h_h.at[src_v.at[j]], rs_v.at[b], gf_sem[b])
            pltpu.async_copy(h_h.at[dst_v.at[j]], rd_v.at[b], gf_sem[b])

        def wait_gf(b):
            pltpu.make_async_copy(h_h.at[src_v.at[0]], rs_v.at[b], gf_sem[b]).wait()
            pltpu.make_async_copy(h_h.at[dst_v.at[0]], rd_v.at[b], gf_sem[b]).wait()

        def wait_w(b):
            pltpu.make_async_copy(rs_v.at[b], out_h.at[wid, 0, :, fsl],
                                  w_sem[b]).wait()

        for b in range(NBUF - 1):
            issue_gf(b, b)

        def round_body(r, carry):
            i0 = r * NBUF
            for b in range(NBUF):
                i = i0 + b
                bj = (b + NBUF - 1) % NBUF
                j = i + NBUF - 1

                @pl.when(j < N_CHUNKS)
                def _prefetch():
                    @pl.when(j >= NBUF)
                    def _drain():
                        wait_w(bj)

                    issue_gf(j, bj)

                wait_gf(b)
                _mul_rows(rs_v.at[b], rd_v.at[b])
                pltpu.async_copy(rs_v.at[b], out_h.at[wid, i, :, fsl], w_sem[b])
            return carry

        lax.fori_loop(0, N_ROUNDS, round_body, 0)
        for b in range(NBUF):
            wait_w(b)

    one_pass(hlo_h, out4_h, 0)
    one_pass(hhi_h, out4_h, 1)


def _pair_products(hs, src3, dst3):
    k = functools.partial(
        pl.kernel,
        out_type=jax.ShapeDtypeStruct((NW, N_CHUNKS, CHUNK, HID_C),
                                      jnp.float32),
        mesh=_MESH,
        scratch_types=[
            pltpu.VMEM((N_CHUNKS, CHUNK), jnp.int32),
            pltpu.VMEM((N_CHUNKS, CHUNK), jnp.int32),
            pltpu.VMEM((NBUF, CHUNK, HALF), jnp.float32),
            pltpu.VMEM((NBUF, CHUNK, HALF), jnp.float32),
        ] + [pltpu.SemaphoreType.DMA] * (2 * NBUF),
        compiler_params=pltpu.CompilerParams(use_tc_tiling_on_sc=False),
    )(_pair_body)
    return k(hs[0], hs[1], src3, dst3)


def _final_body(hhg_ref, hhl_ref, eag_ref, eal_ref,
                w1ag_ref, w1bg_ref, b1g_ref, w2g_ref, b2g_ref, w3g_ref, b3g_ref,
                w1al_ref, w1bl_ref, b1l_ref, w2l_ref, b2l_ref, w3l_ref, b3l_ref,
                out_ref):
    def branch(hh_ref, ea_ref, w1a, w1b, b1, w2, b2, w3, b3):
        hh = hh_ref[...].reshape(EB, HID_C)
        x = _silu(jnp.dot(hh, w1a[...], preferred_element_type=jnp.float32)
                  + jnp.dot(ea_ref[...], w1b[...], preferred_element_type=jnp.float32)
                  + b1[...])
        x = _silu(jnp.dot(x, w2[...], preferred_element_type=jnp.float32) + b2[...])
        return jnp.dot(x, w3[...], preferred_element_type=jnp.float32) + b3[...]

    og = branch(hhg_ref, eag_ref, w1ag_ref, w1bg_ref, b1g_ref, w2g_ref,
                b2g_ref, w3g_ref, b3g_ref)
    ol = branch(hhl_ref, eal_ref, w1al_ref, w1bl_ref, b1l_ref, w2l_ref,
                b2l_ref, w3l_ref, b3l_ref)
    out_ref[...] = jnp.concatenate([og, ol], axis=1)


def _final_mlp(hh_g, hh_l, ea_g, ea_l,
               Wm1_g, bm1_g, Wm2_g, bm2_g, Wm3_g, bm3_g,
               Wm1_l, bm1_l, Wm2_l, bm2_l, Wm3_l, bm3_l):
    w_spec = lambda shape: pl.BlockSpec(shape, lambda w, g: (0, 0))
    h_spec = pl.BlockSpec((1, GB, CHUNK, HID_C), lambda w, g: (w, g, 0, 0))
    e_spec = pl.BlockSpec((EB, HID_C), lambda w, g: (NG * w + g, 0))
    return pl.pallas_call(
        _final_body,
        grid=(NW, NG),
        in_specs=[
            h_spec, h_spec, e_spec, e_spec,
            w_spec((HID_C, HID_C)), w_spec((HID_C, HID_C)), w_spec((1, HID_C)),
            w_spec((HID_C, 64)), w_spec((1, 64)), w_spec((64, 1)), w_spec((1, 1)),
            w_spec((HID_C, HID_C)), w_spec((HID_C, HID_C)), w_spec((1, HID_C)),
            w_spec((HID_C, 64)), w_spec((1, 64)), w_spec((64, 1)), w_spec((1, 1)),
        ],
        out_specs=pl.BlockSpec((EB, 2), lambda w, g: (NG * w + g, 0)),
        out_shape=jax.ShapeDtypeStruct((N_EDGES_C, 2), jnp.float32),
    )(hh_g, hh_l, ea_g, ea_l,
      Wm1_g[:HID_C], Wm1_g[HID_C:], bm1_g.reshape(1, HID_C),
      Wm2_g, bm2_g.reshape(1, 64), Wm3_g, bm3_g.reshape(1, 1),
      Wm1_l[:HID_C], Wm1_l[HID_C:], bm1_l.reshape(1, HID_C),
      Wm2_l, bm2_l.reshape(1, 64), Wm3_l, bm3_l.reshape(1, 1))


def kernel(atom_type, r_feat, p_feat, pos, bond_index, bond_type, batch,
           atom_tab, W_feat,
           We1_g, be1_g, We2_g, be2_g, bond_g, Wconv_g, bconv_g,
           Wm1_g, bm1_g, Wm2_g, bm2_g, Wm3_g, bm3_g,
           We1_l, be1_l, We2_l, be2_l, bond_l, Wconv_l, bconv_l,
           Wm1_l, bm1_l, Wm2_l, bm2_l, Wm3_l, bm3_l):
    src = bond_index[0].astype(jnp.int32)
    dst = bond_index[1].astype(jnp.int32)
    src3 = src.reshape(NW, N_CHUNKS, CHUNK)
    dst3 = dst.reshape(NW, N_CHUNKS, CHUNK)
    zeros_block = jnp.zeros((RTA, HALF), jnp.float32)

    z = _node_embed(atom_type, r_feat, p_feat, atom_tab, W_feat)
    d2 = _edge_d2(pos, src, dst)
    ea_g, ea_l = _edge_attr(d2, bond_type, We1_g, be1_g, We2_g, be2_g, bond_g,
                            We1_l, be1_l, We2_l, be2_l, bond_l)

    fg_0, fg_1 = _filters_branch(ea_g, Wconv_g, bconv_g)
    fl_0, fl_1 = _filters_branch(ea_l, Wconv_l, bconv_l)

    hs_g = z
    hs_l = z
    for l, (f_g, f_l) in enumerate(((fg_0, fl_0), (fg_1, fl_1))):
        agg_g = _gather_mul_scatter(hs_g[0], hs_g[1], f_g, src3, dst3,
                                    zeros_block)
        hs_g = _node_update(hs_g, agg_g, Wconv_g[l, 2], bconv_g[l, 2])
        agg_l = _gather_mul_scatter(hs_l[0], hs_l[1], f_l, src3, dst3,
                                    zeros_block)
        hs_l = _node_update(hs_l, agg_l, Wconv_l[l, 2], bconv_l[l, 2])

    hh_g = _pair_products(hs_g, src3, dst3)
    hh_l = _pair_products(hs_l, src3, dst3)
    return _final_mlp(hh_g, hh_l, ea_g, ea_l,
                      Wm1_g, bm1_g, Wm2_g, bm2_g, Wm3_g, bm3_g,
                      Wm1_l, bm1_l, Wm2_l, bm2_l, Wm3_l, bm3_l)


# speedup vs baseline: 1.6755x; 1.0146x over previous
import functools

import jax
import jax.numpy as jnp
from jax import lax
from jax.experimental import pallas as pl
from jax.experimental.pallas import tpu as pltpu
from jax.experimental.pallas import tpu_sc as plsc

N_NODES_C = 10000
N_EDGES_C = 320000
HID_C = 128
HALF = 64
NUM_BOND_C = 24
NUM_ATOM_C = 100

NC = 2
NS = 16
NW = NC * NS
EPW = N_EDGES_C // NW
CHUNK = 80
N_CHUNKS = EPW // CHUNK
NBUF = 5
N_ROUNDS = N_CHUNKS // NBUF
RTA = 624
TAIL = N_NODES_C - NS * RTA

_MESH = plsc.VectorSubcoreMesh(core_axis_name="c", subcore_axis_name="s")


def _silu(x):
    return x * (1.0 / (1.0 + jnp.exp(-x)))


def _mul_rows(rows, other):
    def mulrow(r, cc):
        for j in range(HALF // 16):
            sl = pl.ds(j * 16, 16)
            rows[r, sl] = rows[r, sl] * other[r, sl]
        return cc

    lax.fori_loop(0, CHUNK, mulrow, 0)


def _node_embed_body(at_ref, rf_ref, pf_ref, tab_ref, wf_ref, zlo_ref, zhi_ref):
    oh = (at_ref[...] == lax.broadcasted_iota(jnp.int32, (at_ref.shape[0], NUM_ATOM_C), 1)
          ).astype(jnp.float32)
    emb = jnp.dot(oh, tab_ref[...], preferred_element_type=jnp.float32)
    fr = jnp.dot(rf_ref[...], wf_ref[...], preferred_element_type=jnp.float32)
    fp = jnp.dot(pf_ref[...], wf_ref[...], preferred_element_type=jnp.float32)
    zlo_ref[...] = emb + fr
    zhi_ref[...] = fp - fr


def _node_embed(atom_type, r_feat, p_feat, atom_tab, W_feat):
    nb = 1000
    grid = N_NODES_C // nb
    return pl.pallas_call(
        _node_embed_body,
        grid=(grid,),
        in_specs=[
            pl.BlockSpec((nb, 1), lambda i: (i, 0)),
            pl.BlockSpec((nb, 128), lambda i: (i, 0)),
            pl.BlockSpec((nb, 128), lambda i: (i, 0)),
            pl.BlockSpec((NUM_ATOM_C, HALF), lambda i: (0, 0)),
            pl.BlockSpec((128, HALF), lambda i: (0, 0)),
        ],
        out_specs=[pl.BlockSpec((nb, HALF), lambda i: (i, 0))] * 2,
        out_shape=[jax.ShapeDtypeStruct((N_NODES_C, HALF), jnp.float32)] * 2,
    )(atom_type.reshape(N_NODES_C, 1).astype(jnp.int32), r_feat, p_feat,
      atom_tab, W_feat)


D2CH = 2000


def _d2_sc_body(px_h, py_h, pz_h, src_h, dst_h, bt_h, d2p_h,
                px_v, py_v, pz_v, src_v, dst_v, bt_v, flat_v):
    wid = lax.axis_index("s") * NC + lax.axis_index("c")
    base = wid * EPW
    pltpu.sync_copy(px_h, px_v)
    pltpu.sync_copy(py_h, py_v)
    pltpu.sync_copy(pz_h, pz_v)
    pltpu.sync_copy(src_h.at[pl.ds(base, EPW)], src_v)
    pltpu.sync_copy(dst_h.at[pl.ds(base, EPW)], dst_v)
    pltpu.sync_copy(bt_h.at[pl.ds(base, EPW)], bt_v)

    lane = lax.iota(jnp.int32, 16)
    zeros16 = jnp.zeros((16,), jnp.int32)

    def chunk(c, carry):
        def body(i, cc):
            sl = pl.ds(c * D2CH + i * 16, 16)
            s = src_v[sl]
            t = dst_v[sl]
            dx = plsc.load_gather(px_v, [t]) - plsc.load_gather(px_v, [s])
            dy = plsc.load_gather(py_v, [t]) - plsc.load_gather(py_v, [s])
            dz = plsc.load_gather(pz_v, [t]) - plsc.load_gather(pz_v, [s])
            row = i * 16 + lane
            plsc.store_scatter(flat_v, [row, zeros16],
                               dx * dx + dy * dy + dz * dz)
            plsc.store_scatter(flat_v, [row, zeros16 + 1],
                               bt_v[sl].astype(jnp.float32))
            return cc

        lax.fori_loop(0, D2CH // 16, body, 0)
        pltpu.sync_copy(flat_v,
                        d2p_h.at[pl.ds(base + c * D2CH, D2CH), pl.ds(0, 16)])
        return carry

    lax.fori_loop(0, EPW // D2CH, chunk, 0)


def _edge_d2(pos, src, dst, bond_type):
    k = functools.partial(
        pl.kernel,
        out_type=jax.ShapeDtypeStruct((N_EDGES_C, HID_C), jnp.float32),
        mesh=_MESH,
        scratch_types=[
            pltpu.VMEM((N_NODES_C,), jnp.float32),
            pltpu.VMEM((N_NODES_C,), jnp.float32),
            pltpu.VMEM((N_NODES_C,), jnp.float32),
            pltpu.VMEM((EPW,), jnp.int32),
            pltpu.VMEM((EPW,), jnp.int32),
            pltpu.VMEM((EPW,), jnp.int32),
            pltpu.VMEM((D2CH, 16), jnp.float32),
        ],
        compiler_params=pltpu.CompilerParams(needs_layout_passes=False,
                                             use_tc_tiling_on_sc=False),
    )(_d2_sc_body)
    pos_t = pos.T
    return k(pos_t[0], pos_t[1], pos_t[2], src, dst, bond_type.astype(jnp.int32))


def _edge_attr_body(d2p_ref, we1_ref, be1_ref, we2_ref, be2_ref, bond_ref,
                    ea_ref):
    eb = ea_ref.shape[0]
    blk = d2p_ref[...]
    d = jnp.sqrt(blk[:, 0:1])
    bti = blk[:, 1:2].astype(jnp.int32)
    oh = (bti == lax.broadcasted_iota(jnp.int32, (eb, NUM_BOND_C), 1)
          ).astype(jnp.float32)
    e1 = _silu(d * we1_ref[...] + be1_ref[...])
    e = jnp.dot(e1, we2_ref[...], preferred_element_type=jnp.float32) + be2_ref[...]
    bemb = jnp.dot(oh, bond_ref[...], preferred_element_type=jnp.float32)
    ea_ref[...] = e * bemb


def _edge_attr_branch(d2p, We1, be1, We2, be2, bond):
    eb = 2560
    grid = N_EDGES_C // eb
    w_spec = lambda shape: pl.BlockSpec(shape, lambda i: (0, 0))
    return pl.pallas_call(
        _edge_attr_body,
        grid=(grid,),
        in_specs=[
            pl.BlockSpec((eb, HID_C), lambda i: (i, 0)),
            w_spec((1, HID_C)), w_spec((1, HID_C)), w_spec((HID_C, HID_C)),
            w_spec((1, HID_C)), w_spec((NUM_BOND_C, HID_C)),
        ],
        out_specs=pl.BlockSpec((eb, HID_C), lambda i: (i, 0)),
        out_shape=jax.ShapeDtypeStruct((N_EDGES_C, HID_C), jnp.float32),
    )(d2p, We1, be1.reshape(1, HID_C), We2, be2.reshape(1, HID_C), bond)


GB = 25
EB = GB * CHUNK
NG = N_CHUNKS // GB


def _filters_body(ea_ref, wf1_ref, bf1_ref, wf2_ref, bf2_ref, f_ref):
    t = _silu(jnp.dot(ea_ref[...], wf1_ref[...],
                      preferred_element_type=jnp.float32) + bf1_ref[...])
    f = jnp.dot(t, wf2_ref[...], preferred_element_type=jnp.float32) + bf2_ref[...]
    f_ref[...] = f.reshape(1, GB, CHUNK, HID_C)


def _filters_one(ea, Wconv, bconv, l):
    w_spec = lambda shape: pl.BlockSpec(shape, lambda w, g: (0, 0))
    e_spec = pl.BlockSpec((EB, HID_C), lambda w, g: (NG * w + g, 0))
    f_spec = pl.BlockSpec((1, GB, CHUNK, HID_C), lambda w, g: (w, g, 0, 0))
    return pl.pallas_call(
        _filters_body,
        grid=(NW, NG),
        in_specs=[
            e_spec,
            w_spec((HID_C, HID_C)), w_spec((1, HID_C)),
            w_spec((HID_C, HID_C)), w_spec((1, HID_C)),
        ],
        out_specs=f_spec,
        out_shape=jax.ShapeDtypeStruct((NW, N_CHUNKS, CHUNK, HID_C),
                                       jnp.float32),
    )(ea,
      Wconv[l, 0], bconv[l, 0].reshape(1, HID_C),
      Wconv[l, 1], bconv[l, 1].reshape(1, HID_C))


def _gms_body(hlo_h, hhi_h, f_h, src_h, dst_h, zeros_h, out_h,
              src_v, dst_v, rows_v, filt_v, agg_sh, *sems):
    gf_sem = sems[:NBUF]
    s_sem = sems[NBUF:]
    c = lax.axis_index("c")
    s = lax.axis_index("s")
    wid = s * NC + c

    pltpu.sync_copy(src_h.at[wid], src_v)
    pltpu.sync_copy(dst_h.at[wid], dst_v)

    for half, h_h in enumerate((hlo_h, hhi_h)):
        fsl = pl.ds(half * HALF, HALF)

        def issue_gf(j, b):
            pltpu.async_copy(h_h.at[src_v.at[j]], rows_v.at[b], gf_sem[b])
            pltpu.async_copy(f_h.at[wid, j, :, fsl], filt_v.at[b], gf_sem[b])

        def wait_gf(b):
            pltpu.make_async_copy(h_h.at[src_v.at[0]], rows_v.at[b],
                                  gf_sem[b]).wait()
            pltpu.make_async_copy(f_h.at[wid, 0, :, fsl], filt_v.at[b],
                                  gf_sem[b]).wait()

        def wait_scatter(b):
            pltpu.make_async_copy(rows_v.at[b], agg_sh.at[dst_v.at[0]],
                                  s_sem[b]).wait()

        for b in range(NBUF - 1):
            issue_gf(b, b)

        pltpu.sync_copy(zeros_h, agg_sh.at[pl.ds(s * RTA, RTA)])

        @pl.when(s == 0)
        def _zero_tail():
            pltpu.sync_copy(zeros_h.at[pl.ds(0, TAIL)],
                            agg_sh.at[pl.ds(NS * RTA, TAIL)])

        plsc.subcore_barrier()

        def round_body(r, carry):
            i0 = r * NBUF
            for b in range(NBUF):
                i = i0 + b
                bj = (b + NBUF - 1) % NBUF
                j = i + NBUF - 1

                @pl.when(j < N_CHUNKS)
                def _prefetch():
                    @pl.when(j >= NBUF)
                    def _drain():
                        wait_scatter(bj)

                    issue_gf(j, bj)

                wait_gf(b)
                _mul_rows(rows_v.at[b], filt_v.at[b])
                pltpu.async_copy(rows_v.at[b], agg_sh.at[dst_v.at[i]], s_sem[b],
                                 add=True)
            return carry

        lax.fori_loop(0, N_ROUNDS, round_body, 0)
        for b in range(NBUF):
            wait_scatter(b)
        plsc.subcore_barrier()
        pltpu.sync_copy(agg_sh.at[pl.ds(s * RTA, RTA)],
                        out_h.at[c, half, pl.ds(s * RTA, RTA)])

        @pl.when(s == 0)
        def _copy_tail():
            pltpu.sync_copy(agg_sh.at[pl.ds(NS * RTA, TAIL)],
                            out_h.at[c, half, pl.ds(NS * RTA, TAIL)])

        plsc.subcore_barrier()


def _gather_mul_scatter(h_lo, h_hi, f4, src3, dst3, zeros_block):
    k = functools.partial(
        pl.kernel,
        out_type=jax.ShapeDtypeStruct((NC, 2, N_NODES_C, HALF), jnp.float32),
        mesh=_MESH,
        scratch_types=[
            pltpu.VMEM((N_CHUNKS, CHUNK), jnp.int32),
            pltpu.VMEM((N_CHUNKS, CHUNK), jnp.int32),
            pltpu.VMEM((NBUF, CHUNK, HALF), jnp.float32),
            pltpu.VMEM((NBUF, CHUNK, HALF), jnp.float32),
            pltpu.VMEM_SHARED((N_NODES_C, HALF), jnp.float32),
        ] + [pltpu.SemaphoreType.DMA] * (2 * NBUF),
        compiler_params=pltpu.CompilerParams(use_tc_tiling_on_sc=False),
    )(_gms_body)
    return k(h_lo, h_hi, f4, src3, dst3, zeros_block)


def _update_body(hlo_ref, hhi_ref, a_ref, wu_ref, bu_ref, olo_ref, ohi_ref):
    agg = jnp.concatenate([a_ref[0, 0] + a_ref[1, 0],
                           a_ref[0, 1] + a_ref[1, 1]], axis=1)
    x = _silu(jnp.dot(agg, wu_ref[...], preferred_element_type=jnp.float32)
              + bu_ref[...])
    olo_ref[...] = hlo_ref[...] + x[:, :HALF]
    ohi_ref[...] = hhi_ref[...] + x[:, HALF:]


def _node_update(hs, agg, wu, bu):
    nb = 1000
    grid = N_NODES_C // nb
    w_spec = lambda shape: pl.BlockSpec(shape, lambda i: tuple([0] * len(shape)))
    h_spec = pl.BlockSpec((nb, HALF), lambda i: (i, 0))
    a_spec = pl.BlockSpec((NC, 2, nb, HALF), lambda i: (0, 0, i, 0))
    out = pl.pallas_call(
        _update_body,
        grid=(grid,),
        in_specs=[h_spec, h_spec, a_spec,
                  w_spec((HID_C, HID_C)), w_spec((1, HID_C))],
        out_specs=[h_spec] * 2,
        out_shape=[jax.ShapeDtypeStruct((N_NODES_C, HALF), jnp.float32)] * 2,
    )(hs[0], hs[1], agg, wu, bu.reshape(1, HID_C))
    return (out[0], out[1])


def _pair_body(hlo_h, hhi_h, src_h, dst_h, out4_h,
               src_v, dst_v, rs_v, rd_v, *sems):
    gf_sem = sems[:NBUF]
    w_sem = sems[NBUF:]
    wid = lax.axis_index("s") * NC + lax.axis_index("c")

    pltpu.sync_copy(src_h.at[wid], src_v)
    pltpu.sync_copy(dst_h.at[wid], dst_v)

    def one_pass(h_h, out_h, half):
        fsl = pl.ds(half * HALF, HALF)

        def issue_gf(j, b):
            pltpu.async_copy(h_h.at[src_v.at[j]], rs_v.at[b], gf_sem[b])
            pltpu.async_copy(h_h.at[dst_v.at[j]], rd_v.at[b], gf_sem[b])

        def wait_gf(b):
            pltpu.make_async_copy(h_h.at[src_v.at[0]], rs_v.at[b], gf_sem[b]).wait()
            pltpu.make_async_copy(h_h.at[dst_v.at[0]], rd_v.at[b], gf_sem[b]).wait()

        def wait_w(b):
            pltpu.make_async_copy(rs_v.at[b], out_h.at[wid, 0, :, fsl],
                                  w_sem[b]).wait()

        for b in range(NBUF - 1):
            issue_gf(b, b)

        def round_body(r, carry):
            i0 = r * NBUF
            for b in range(NBUF):
                i = i0 + b
                bj = (b + NBUF - 1) % NBUF
                j = i + NBUF - 1

                @pl.when(j < N_CHUNKS)
                def _prefetch():
                    @pl.when(j >= NBUF)
                    def _drain():
                        wait_w(bj)

                    issue_gf(j, bj)

                wait_gf(b)
                _mul_rows(rs_v.at[b], rd_v.at[b])
                pltpu.async_copy(rs_v.at[b], out_h.at[wid, i, :, fsl], w_sem[b])
            return carry

        lax.fori_loop(0, N_ROUNDS, round_body, 0)
        for b in range(NBUF):
            wait_w(b)

    one_pass(hlo_h, out4_h, 0)
    one_pass(hhi_h, out4_h, 1)


def _pair_products(hs, src3, dst3):
    k = functools.partial(
        pl.kernel,
        out_type=jax.ShapeDtypeStruct((NW, N_CHUNKS, CHUNK, HID_C),
                                      jnp.float32),
        mesh=_MESH,
        scratch_types=[
            pltpu.VMEM((N_CHUNKS, CHUNK), jnp.int32),
            pltpu.VMEM((N_CHUNKS, CHUNK), jnp.int32),
            pltpu.VMEM((NBUF, CHUNK, HALF), jnp.float32),
            pltpu.VMEM((NBUF, CHUNK, HALF), jnp.float32),
        ] + [pltpu.SemaphoreType.DMA] * (2 * NBUF),
        compiler_params=pltpu.CompilerParams(use_tc_tiling_on_sc=False),
    )(_pair_body)
    return k(hs[0], hs[1], src3, dst3)


def _final_body(hhg_ref, hhl_ref, eag_ref, eal_ref,
                w1ag_ref, w1bg_ref, b1g_ref, w2g_ref, b2g_ref, w3g_ref, b3g_ref,
                w1al_ref, w1bl_ref, b1l_ref, w2l_ref, b2l_ref, w3l_ref, b3l_ref,
                out_ref):
    def branch(hh_ref, ea_ref, w1a, w1b, b1, w2, b2, w3, b3):
        hh = hh_ref[...].reshape(EB, HID_C)
        x = _silu(jnp.dot(hh, w1a[...], preferred_element_type=jnp.float32)
                  + jnp.dot(ea_ref[...], w1b[...], preferred_element_type=jnp.float32)
                  + b1[...])
        x = _silu(jnp.dot(x, w2[...], preferred_element_type=jnp.float32) + b2[...])
        return jnp.dot(x, w3[...], preferred_element_type=jnp.float32) + b3[...]

    og = branch(hhg_ref, eag_ref, w1ag_ref, w1bg_ref, b1g_ref, w2g_ref,
                b2g_ref, w3g_ref, b3g_ref)
    ol = branch(hhl_ref, eal_ref, w1al_ref, w1bl_ref, b1l_ref, w2l_ref,
                b2l_ref, w3l_ref, b3l_ref)
    out_ref[...] = jnp.concatenate([og, ol], axis=1)


def _final_mlp(hh_g, hh_l, ea_g, ea_l,
               Wm1_g, bm1_g, Wm2_g, bm2_g, Wm3_g, bm3_g,
               Wm1_l, bm1_l, Wm2_l, bm2_l, Wm3_l, bm3_l):
    w_spec = lambda shape: pl.BlockSpec(shape, lambda w, g: (0, 0))
    h_spec = pl.BlockSpec((1, GB, CHUNK, HID_C), lambda w, g: (w, g, 0, 0))
    e_spec = pl.BlockSpec((EB, HID_C), lambda w, g: (NG * w + g, 0))
    return pl.pallas_call(
        _final_body,
        grid=(NW, NG),
        in_specs=[
            h_spec, h_spec, e_spec, e_spec,
            w_spec((HID_C, HID_C)), w_spec((HID_C, HID_C)), w_spec((1, HID_C)),
            w_spec((HID_C, 64)), w_spec((1, 64)), w_spec((64, 1)), w_spec((1, 1)),
            w_spec((HID_C, HID_C)), w_spec((HID_C, HID_C)), w_spec((1, HID_C)),
            w_spec((HID_C, 64)), w_spec((1, 64)), w_spec((64, 1)), w_spec((1, 1)),
        ],
        out_specs=pl.BlockSpec((EB, 2), lambda w, g: (NG * w + g, 0)),
        out_shape=jax.ShapeDtypeStruct((N_EDGES_C, 2), jnp.float32),
    )(hh_g, hh_l, ea_g, ea_l,
      Wm1_g[:HID_C], Wm1_g[HID_C:], bm1_g.reshape(1, HID_C),
      Wm2_g, bm2_g.reshape(1, 64), Wm3_g, bm3_g.reshape(1, 1),
      Wm1_l[:HID_C], Wm1_l[HID_C:], bm1_l.reshape(1, HID_C),
      Wm2_l, bm2_l.reshape(1, 64), Wm3_l, bm3_l.reshape(1, 1))


def kernel(atom_type, r_feat, p_feat, pos, bond_index, bond_type, batch,
           atom_tab, W_feat,
           We1_g, be1_g, We2_g, be2_g, bond_g, Wconv_g, bconv_g,
           Wm1_g, bm1_g, Wm2_g, bm2_g, Wm3_g, bm3_g,
           We1_l, be1_l, We2_l, be2_l, bond_l, Wconv_l, bconv_l,
           Wm1_l, bm1_l, Wm2_l, bm2_l, Wm3_l, bm3_l):
    src = bond_index[0].astype(jnp.int32)
    dst = bond_index[1].astype(jnp.int32)
    src3 = src.reshape(NW, N_CHUNKS, CHUNK)
    dst3 = dst.reshape(NW, N_CHUNKS, CHUNK)
    zeros_block = jnp.zeros((RTA, HALF), jnp.float32)

    z = _node_embed(atom_type, r_feat, p_feat, atom_tab, W_feat)
    d2p = _edge_d2(pos, src, dst, bond_type)

    ea_g = _edge_attr_branch(d2p, We1_g, be1_g, We2_g, be2_g, bond_g)
    fg_0 = _filters_one(ea_g, Wconv_g, bconv_g, 0)
    ea_l = _edge_attr_branch(d2p, We1_l, be1_l, We2_l, be2_l, bond_l)
    fl_0 = _filters_one(ea_l, Wconv_l, bconv_l, 0)
    fg_1 = _filters_one(ea_g, Wconv_g, bconv_g, 1)
    fl_1 = _filters_one(ea_l, Wconv_l, bconv_l, 1)

    hs_g = z
    hs_l = z
    for l, (f_g, f_l) in enumerate(((fg_0, fl_0), (fg_1, fl_1))):
        agg_g = _gather_mul_scatter(hs_g[0], hs_g[1], f_g, src3, dst3,
                                    zeros_block)
        agg_l = _gather_mul_scatter(hs_l[0], hs_l[1], f_l, src3, dst3,
                                    zeros_block)
        hs_g = _node_update(hs_g, agg_g, Wconv_g[l, 2], bconv_g[l, 2])
        hs_l = _node_update(hs_l, agg_l, Wconv_l[l, 2], bconv_l[l, 2])

    hh_g = _pair_products(hs_g, src3, dst3)
    hh_l = _pair_products(hs_l, src3, dst3)
    return _final_mlp(hh_g, hh_l, ea_g, ea_l,
                      Wm1_g, bm1_g, Wm2_g, bm2_g, Wm3_g, bm3_g,
                      Wm1_l, bm1_l, Wm2_l, bm2_l, Wm3_l, bm3_l)


# speedup vs baseline: 1.8684x; 1.1151x over previous
import functools

import jax
import jax.numpy as jnp
from jax import lax
from jax.experimental import pallas as pl
from jax.experimental.pallas import tpu as pltpu
from jax.experimental.pallas import tpu_sc as plsc

N_NODES_C = 10000
N_EDGES_C = 320000
HID_C = 128
HALF = 64
NUM_BOND_C = 24
NUM_ATOM_C = 100

NC = 2
NS = 16
NW = NC * NS
EPW = N_EDGES_C // NW
CHUNK = 80
N_CHUNKS = EPW // CHUNK
NBUF = 5
N_ROUNDS = N_CHUNKS // NBUF
RTA = 624
TAIL = N_NODES_C - NS * RTA

_MESH = plsc.VectorSubcoreMesh(core_axis_name="c", subcore_axis_name="s")


def _silu(x):
    return x * (1.0 / (1.0 + jnp.exp(-x)))


def _mul_rows(rows, other):
    def mulrow(r, cc):
        for j in range(HALF // 16):
            sl = pl.ds(j * 16, 16)
            rows[r, sl] = rows[r, sl] * other[r, sl]
        return cc

    lax.fori_loop(0, CHUNK, mulrow, 0)


def _node_embed_body(at_ref, rf_ref, pf_ref, tab_ref, wf_ref, zlo_ref, zhi_ref):
    oh = (at_ref[...] == lax.broadcasted_iota(jnp.int32, (at_ref.shape[0], NUM_ATOM_C), 1)
          ).astype(jnp.float32)
    emb = jnp.dot(oh, tab_ref[...], preferred_element_type=jnp.float32)
    fr = jnp.dot(rf_ref[...], wf_ref[...], preferred_element_type=jnp.float32)
    fp = jnp.dot(pf_ref[...], wf_ref[...], preferred_element_type=jnp.float32)
    zlo_ref[...] = emb + fr
    zhi_ref[...] = fp - fr


def _node_embed(atom_type, r_feat, p_feat, atom_tab, W_feat):
    nb = 1000
    grid = N_NODES_C // nb
    return pl.pallas_call(
        _node_embed_body,
        grid=(grid,),
        in_specs=[
            pl.BlockSpec((nb, 1), lambda i: (i, 0)),
            pl.BlockSpec((nb, 128), lambda i: (i, 0)),
            pl.BlockSpec((nb, 128), lambda i: (i, 0)),
            pl.BlockSpec((NUM_ATOM_C, HALF), lambda i: (0, 0)),
            pl.BlockSpec((128, HALF), lambda i: (0, 0)),
        ],
        out_specs=[pl.BlockSpec((nb, HALF), lambda i: (i, 0))] * 2,
        out_shape=[jax.ShapeDtypeStruct((N_NODES_C, HALF), jnp.float32)] * 2,
    )(atom_type.reshape(N_NODES_C, 1).astype(jnp.int32), r_feat, p_feat,
      atom_tab, W_feat)


D2CH = 2000


def _d2_sc_body(px_h, py_h, pz_h, src_h, dst_h, bt_h, d2p_h,
                px_v, py_v, pz_v, src_v, dst_v, bt_v, flat_v):
    wid = lax.axis_index("s") * NC + lax.axis_index("c")
    base = wid * EPW
    pltpu.sync_copy(px_h, px_v)
    pltpu.sync_copy(py_h, py_v)
    pltpu.sync_copy(pz_h, pz_v)
    pltpu.sync_copy(src_h.at[pl.ds(base, EPW)], src_v)
    pltpu.sync_copy(dst_h.at[pl.ds(base, EPW)], dst_v)
    pltpu.sync_copy(bt_h.at[pl.ds(base, EPW)], bt_v)

    lane = lax.iota(jnp.int32, 16)
    zeros16 = jnp.zeros((16,), jnp.int32)

    def chunk(c, carry):
        def body(i, cc):
            sl = pl.ds(c * D2CH + i * 16, 16)
            s = src_v[sl]
            t = dst_v[sl]
            dx = plsc.load_gather(px_v, [t]) - plsc.load_gather(px_v, [s])
            dy = plsc.load_gather(py_v, [t]) - plsc.load_gather(py_v, [s])
            dz = plsc.load_gather(pz_v, [t]) - plsc.load_gather(pz_v, [s])
            row = i * 16 + lane
            plsc.store_scatter(flat_v, [row, zeros16],
                               dx * dx + dy * dy + dz * dz)
            plsc.store_scatter(flat_v, [row, zeros16 + 1],
                               bt_v[sl].astype(jnp.float32))
            return cc

        lax.fori_loop(0, D2CH // 16, body, 0)
        pltpu.sync_copy(flat_v,
                        d2p_h.at[pl.ds(base + c * D2CH, D2CH), pl.ds(0, 16)])
        return carry

    lax.fori_loop(0, EPW // D2CH, chunk, 0)


def _edge_d2(pos, src, dst, bond_type):
    k = functools.partial(
        pl.kernel,
        out_type=jax.ShapeDtypeStruct((N_EDGES_C, HID_C), jnp.float32),
        mesh=_MESH,
        scratch_types=[
            pltpu.VMEM((N_NODES_C,), jnp.float32),
            pltpu.VMEM((N_NODES_C,), jnp.float32),
            pltpu.VMEM((N_NODES_C,), jnp.float32),
            pltpu.VMEM((EPW,), jnp.int32),
            pltpu.VMEM((EPW,), jnp.int32),
            pltpu.VMEM((EPW,), jnp.int32),
            pltpu.VMEM((D2CH, 16), jnp.float32),
        ],
        compiler_params=pltpu.CompilerParams(needs_layout_passes=False,
                                             use_tc_tiling_on_sc=False),
    )(_d2_sc_body)
    pos_t = pos.T
    return k(pos_t[0], pos_t[1], pos_t[2], src, dst, bond_type.astype(jnp.int32))


GB = 25
EB = GB * CHUNK
NG = N_CHUNKS // GB


def _ea_filt0_body(d2p_ref, we1_ref, be1_ref, we2_ref, be2_ref, bond_ref,
                   wf1_ref, bf1_ref, wf2_ref, bf2_ref, ea_ref, f_ref):
    blk = d2p_ref[...]
    d = jnp.sqrt(blk[:, 0:1])
    bti = blk[:, 1:2].astype(jnp.int32)
    oh = (bti == lax.broadcasted_iota(jnp.int32, (EB, NUM_BOND_C), 1)
          ).astype(jnp.float32)
    e1 = _silu(d * we1_ref[...] + be1_ref[...])
    e = jnp.dot(e1, we2_ref[...], preferred_element_type=jnp.float32) + be2_ref[...]
    bemb = jnp.dot(oh, bond_ref[...], preferred_element_type=jnp.float32)
    ea = e * bemb
    ea_ref[...] = ea
    t = _silu(jnp.dot(ea, wf1_ref[...], preferred_element_type=jnp.float32)
              + bf1_ref[...])
    f = jnp.dot(t, wf2_ref[...], preferred_element_type=jnp.float32) + bf2_ref[...]
    f_ref[...] = f.reshape(1, GB, CHUNK, HID_C)


def _ea_filt0_branch(d2p, We1, be1, We2, be2, bond, Wconv, bconv):
    w_spec = lambda shape: pl.BlockSpec(shape, lambda w, g: (0, 0))
    e_spec = pl.BlockSpec((EB, HID_C), lambda w, g: (NG * w + g, 0))
    f_spec = pl.BlockSpec((1, GB, CHUNK, HID_C), lambda w, g: (w, g, 0, 0))
    return pl.pallas_call(
        _ea_filt0_body,
        grid=(NW, NG),
        in_specs=[
            e_spec,
            w_spec((1, HID_C)), w_spec((1, HID_C)), w_spec((HID_C, HID_C)),
            w_spec((1, HID_C)), w_spec((NUM_BOND_C, HID_C)),
            w_spec((HID_C, HID_C)), w_spec((1, HID_C)),
            w_spec((HID_C, HID_C)), w_spec((1, HID_C)),
        ],
        out_specs=[e_spec, f_spec],
        out_shape=[jax.ShapeDtypeStruct((N_EDGES_C, HID_C), jnp.float32),
                   jax.ShapeDtypeStruct((NW, N_CHUNKS, CHUNK, HID_C),
                                        jnp.float32)],
    )(d2p, We1, be1.reshape(1, HID_C), We2, be2.reshape(1, HID_C), bond,
      Wconv[0, 0], bconv[0, 0].reshape(1, HID_C),
      Wconv[0, 1], bconv[0, 1].reshape(1, HID_C))


def _filters_body(ea_ref, wf1_ref, bf1_ref, wf2_ref, bf2_ref, f_ref):
    t = _silu(jnp.dot(ea_ref[...], wf1_ref[...],
                      preferred_element_type=jnp.float32) + bf1_ref[...])
    f = jnp.dot(t, wf2_ref[...], preferred_element_type=jnp.float32) + bf2_ref[...]
    f_ref[...] = f.reshape(1, GB, CHUNK, HID_C)


def _filters_one(ea, Wconv, bconv, l):
    w_spec = lambda shape: pl.BlockSpec(shape, lambda w, g: (0, 0))
    e_spec = pl.BlockSpec((EB, HID_C), lambda w, g: (NG * w + g, 0))
    f_spec = pl.BlockSpec((1, GB, CHUNK, HID_C), lambda w, g: (w, g, 0, 0))
    return pl.pallas_call(
        _filters_body,
        grid=(NW, NG),
        in_specs=[
            e_spec,
            w_spec((HID_C, HID_C)), w_spec((1, HID_C)),
            w_spec((HID_C, HID_C)), w_spec((1, HID_C)),
        ],
        out_specs=f_spec,
        out_shape=jax.ShapeDtypeStruct((NW, N_CHUNKS, CHUNK, HID_C),
                                       jnp.float32),
    )(ea,
      Wconv[l, 0], bconv[l, 0].reshape(1, HID_C),
      Wconv[l, 1], bconv[l, 1].reshape(1, HID_C))


def _gms_body(hlo_h, hhi_h, f_h, src_h, dst_h, zeros_h, out_h,
              src_v, dst_v, rows_v, filt_v, agg_sh, *sems):
    gf_sem = sems[:NBUF]
    s_sem = sems[NBUF:]
    c = lax.axis_index("c")
    s = lax.axis_index("s")
    wid = s * NC + c

    pltpu.sync_copy(src_h.at[wid], src_v)
    pltpu.sync_copy(dst_h.at[wid], dst_v)

    for half, h_h in enumerate((hlo_h, hhi_h)):
        fsl = pl.ds(half * HALF, HALF)

        def issue_gf(j, b):
            pltpu.async_copy(h_h.at[src_v.at[j]], rows_v.at[b], gf_sem[b])
            pltpu.async_copy(f_h.at[wid, j, :, fsl], filt_v.at[b], gf_sem[b])

        def wait_gf(b):
            pltpu.make_async_copy(h_h.at[src_v.at[0]], rows_v.at[b],
                                  gf_sem[b]).wait()
            pltpu.make_async_copy(f_h.at[wid, 0, :, fsl], filt_v.at[b],
                                  gf_sem[b]).wait()

        def wait_scatter(b):
            pltpu.make_async_copy(rows_v.at[b], agg_sh.at[dst_v.at[0]],
                                  s_sem[b]).wait()

        for b in range(NBUF - 1):
            issue_gf(b, b)

        pltpu.sync_copy(zeros_h, agg_sh.at[pl.ds(s * RTA, RTA)])

        @pl.when(s == 0)
        def _zero_tail():
            pltpu.sync_copy(zeros_h.at[pl.ds(0, TAIL)],
                            agg_sh.at[pl.ds(NS * RTA, TAIL)])

        plsc.subcore_barrier()

        def round_body(r, carry):
            i0 = r * NBUF
            for b in range(NBUF):
                i = i0 + b
                bj = (b + NBUF - 1) % NBUF
                j = i + NBUF - 1

                @pl.when(j < N_CHUNKS)
                def _prefetch():
                    @pl.when(j >= NBUF)
                    def _drain():
                        wait_scatter(bj)

                    issue_gf(j, bj)

                wait_gf(b)
                _mul_rows(rows_v.at[b], filt_v.at[b])
                pltpu.async_copy(rows_v.at[b], agg_sh.at[dst_v.at[i]], s_sem[b],
                                 add=True)
            return carry

        lax.fori_loop(0, N_ROUNDS, round_body, 0)
        for b in range(NBUF):
            wait_scatter(b)
        plsc.subcore_barrier()
        pltpu.sync_copy(agg_sh.at[pl.ds(s * RTA, RTA)],
                        out_h.at[c, half, pl.ds(s * RTA, RTA)])

        @pl.when(s == 0)
        def _copy_tail():
            pltpu.sync_copy(agg_sh.at[pl.ds(NS * RTA, TAIL)],
                            out_h.at[c, half, pl.ds(NS * RTA, TAIL)])

        plsc.subcore_barrier()


def _gather_mul_scatter(h_lo, h_hi, f4, src3, dst3, zeros_block):
    k = functools.partial(
        pl.kernel,
        out_type=jax.ShapeDtypeStruct((NC, 2, N_NODES_C, HALF), jnp.float32),
        mesh=_MESH,
        scratch_types=[
            pltpu.VMEM((N_CHUNKS, CHUNK), jnp.int32),
            pltpu.VMEM((N_CHUNKS, CHUNK), jnp.int32),
            pltpu.VMEM((NBUF, CHUNK, HALF), jnp.float32),
            pltpu.VMEM((NBUF, CHUNK, HALF), jnp.float32),
            pltpu.VMEM_SHARED((N_NODES_C, HALF), jnp.float32),
        ] + [pltpu.SemaphoreType.DMA] * (2 * NBUF),
        compiler_params=pltpu.CompilerParams(use_tc_tiling_on_sc=False),
    )(_gms_body)
    return k(h_lo, h_hi, f4, src3, dst3, zeros_block)


def _update_body(hlo_ref, hhi_ref, a_ref, wu_ref, bu_ref, olo_ref, ohi_ref):
    agg = jnp.concatenate([a_ref[0, 0] + a_ref[1, 0],
                           a_ref[0, 1] + a_ref[1, 1]], axis=1)
    x = _silu(jnp.dot(agg, wu_ref[...], preferred_element_type=jnp.float32)
              + bu_ref[...])
    olo_ref[...] = hlo_ref[...] + x[:, :HALF]
    ohi_ref[...] = hhi_ref[...] + x[:, HALF:]


def _node_update(hs, agg, wu, bu):
    nb = 1000
    grid = N_NODES_C // nb
    w_spec = lambda shape: pl.BlockSpec(shape, lambda i: tuple([0] * len(shape)))
    h_spec = pl.BlockSpec((nb, HALF), lambda i: (i, 0))
    a_spec = pl.BlockSpec((NC, 2, nb, HALF), lambda i: (0, 0, i, 0))
    out = pl.pallas_call(
        _update_body,
        grid=(grid,),
        in_specs=[h_spec, h_spec, a_spec,
                  w_spec((HID_C, HID_C)), w_spec((1, HID_C))],
        out_specs=[h_spec] * 2,
        out_shape=[jax.ShapeDtypeStruct((N_NODES_C, HALF), jnp.float32)] * 2,
    )(hs[0], hs[1], agg, wu, bu.reshape(1, HID_C))
    return (out[0], out[1])


def _pair_body(hlo_h, hhi_h, src_h, dst_h, out4_h,
               src_v, dst_v, rs_v, rd_v, *sems):
    gf_sem = sems[:NBUF]
    w_sem = sems[NBUF:]
    wid = lax.axis_index("s") * NC + lax.axis_index("c")

    pltpu.sync_copy(src_h.at[wid], src_v)
    pltpu.sync_copy(dst_h.at[wid], dst_v)

    def one_pass(h_h, out_h, half):
        fsl = pl.ds(half * HALF, HALF)

        def issue_gf(j, b):
            pltpu.async_copy(h_h.at[src_v.at[j]], rs_v.at[b], gf_sem[b])
            pltpu.async_copy(h_h.at[dst_v.at[j]], rd_v.at[b], gf_sem[b])

        def wait_gf(b):
            pltpu.make_async_copy(h_h.at[src_v.at[0]], rs_v.at[b], gf_sem[b]).wait()
            pltpu.make_async_copy(h_h.at[dst_v.at[0]], rd_v.at[b], gf_sem[b]).wait()

        def wait_w(b):
            pltpu.make_async_copy(rs_v.at[b], out_h.at[wid, 0, :, fsl],
                                  w_sem[b]).wait()

        for b in range(NBUF - 1):
            issue_gf(b, b)

        def round_body(r, carry):
            i0 = r * NBUF
            for b in range(NBUF):
                i = i0 + b
                bj = (b + NBUF - 1) % NBUF
                j = i + NBUF - 1

                @pl.when(j < N_CHUNKS)
                def _prefetch():
                    @pl.when(j >= NBUF)
                    def _drain():
                        wait_w(bj)

                    issue_gf(j, bj)

                wait_gf(b)
                _mul_rows(rs_v.at[b], rd_v.at[b])
                pltpu.async_copy(rs_v.at[b], out_h.at[wid, i, :, fsl], w_sem[b])
            return carry

        lax.fori_loop(0, N_ROUNDS, round_body, 0)
        for b in range(NBUF):
            wait_w(b)

    one_pass(hlo_h, out4_h, 0)
    one_pass(hhi_h, out4_h, 1)


def _pair_products(hs, src3, dst3):
    k = functools.partial(
        pl.kernel,
        out_type=jax.ShapeDtypeStruct((NW, N_CHUNKS, CHUNK, HID_C),
                                      jnp.float32),
        mesh=_MESH,
        scratch_types=[
            pltpu.VMEM((N_CHUNKS, CHUNK), jnp.int32),
            pltpu.VMEM((N_CHUNKS, CHUNK), jnp.int32),
            pltpu.VMEM((NBUF, CHUNK, HALF), jnp.float32),
            pltpu.VMEM((NBUF, CHUNK, HALF), jnp.float32),
        ] + [pltpu.SemaphoreType.DMA] * (2 * NBUF),
        compiler_params=pltpu.CompilerParams(use_tc_tiling_on_sc=False),
    )(_pair_body)
    return k(hs[0], hs[1], src3, dst3)


def _final_body(hhg_ref, hhl_ref, eag_ref, eal_ref,
                w1ag_ref, w1bg_ref, b1g_ref, w2g_ref, b2g_ref, w3g_ref, b3g_ref,
                w1al_ref, w1bl_ref, b1l_ref, w2l_ref, b2l_ref, w3l_ref, b3l_ref,
                out_ref):
    def branch(hh_ref, ea_ref, w1a, w1b, b1, w2, b2, w3, b3):
        hh = hh_ref[...].reshape(EB, HID_C)
        x = _silu(jnp.dot(hh, w1a[...], preferred_element_type=jnp.float32)
                  + jnp.dot(ea_ref[...], w1b[...], preferred_element_type=jnp.float32)
                  + b1[...])
        x = _silu(jnp.dot(x, w2[...], preferred_element_type=jnp.float32) + b2[...])
        return jnp.dot(x, w3[...], preferred_element_type=jnp.float32) + b3[...]

    og = branch(hhg_ref, eag_ref, w1ag_ref, w1bg_ref, b1g_ref, w2g_ref,
                b2g_ref, w3g_ref, b3g_ref)
    ol = branch(hhl_ref, eal_ref, w1al_ref, w1bl_ref, b1l_ref, w2l_ref,
                b2l_ref, w3l_ref, b3l_ref)
    out_ref[...] = jnp.concatenate([og, ol], axis=1)


def _final_mlp(hh_g, hh_l, ea_g, ea_l,
               Wm1_g, bm1_g, Wm2_g, bm2_g, Wm3_g, bm3_g,
               Wm1_l, bm1_l, Wm2_l, bm2_l, Wm3_l, bm3_l):
    w_spec = lambda shape: pl.BlockSpec(shape, lambda w, g: (0, 0))
    h_spec = pl.BlockSpec((1, GB, CHUNK, HID_C), lambda w, g: (w, g, 0, 0))
    e_spec = pl.BlockSpec((EB, HID_C), lambda w, g: (NG * w + g, 0))
    return pl.pallas_call(
        _final_body,
        grid=(NW, NG),
        in_specs=[
            h_spec, h_spec, e_spec, e_spec,
            w_spec((HID_C, HID_C)), w_spec((HID_C, HID_C)), w_spec((1, HID_C)),
            w_spec((HID_C, 64)), w_spec((1, 64)), w_spec((64, 1)), w_spec((1, 1)),
            w_spec((HID_C, HID_C)), w_spec((HID_C, HID_C)), w_spec((1, HID_C)),
            w_spec((HID_C, 64)), w_spec((1, 64)), w_spec((64, 1)), w_spec((1, 1)),
        ],
        out_specs=pl.BlockSpec((EB, 2), lambda w, g: (NG * w + g, 0)),
        out_shape=jax.ShapeDtypeStruct((N_EDGES_C, 2), jnp.float32),
    )(hh_g, hh_l, ea_g, ea_l,
      Wm1_g[:HID_C], Wm1_g[HID_C:], bm1_g.reshape(1, HID_C),
      Wm2_g, bm2_g.reshape(1, 64), Wm3_g, bm3_g.reshape(1, 1),
      Wm1_l[:HID_C], Wm1_l[HID_C:], bm1_l.reshape(1, HID_C),
      Wm2_l, bm2_l.reshape(1, 64), Wm3_l, bm3_l.reshape(1, 1))


def kernel(atom_type, r_feat, p_feat, pos, bond_index, bond_type, batch,
           atom_tab, W_feat,
           We1_g, be1_g, We2_g, be2_g, bond_g, Wconv_g, bconv_g,
           Wm1_g, bm1_g, Wm2_g, bm2_g, Wm3_g, bm3_g,
           We1_l, be1_l, We2_l, be2_l, bond_l, Wconv_l, bconv_l,
           Wm1_l, bm1_l, Wm2_l, bm2_l, Wm3_l, bm3_l):
    src = bond_index[0].astype(jnp.int32)
    dst = bond_index[1].astype(jnp.int32)
    src3 = src.reshape(NW, N_CHUNKS, CHUNK)
    dst3 = dst.reshape(NW, N_CHUNKS, CHUNK)
    zeros_block = jnp.zeros((RTA, HALF), jnp.float32)

    z = _node_embed(atom_type, r_feat, p_feat, atom_tab, W_feat)
    d2p = _edge_d2(pos, src, dst, bond_type)

    ea_g, fg_0 = _ea_filt0_branch(d2p, We1_g, be1_g, We2_g, be2_g, bond_g,
                                  Wconv_g, bconv_g)
    ea_l, fl_0 = _ea_filt0_branch(d2p, We1_l, be1_l, We2_l, be2_l, bond_l,
                                  Wconv_l, bconv_l)
    fg_1 = _filters_one(ea_g, Wconv_g, bconv_g, 1)
    fl_1 = _filters_one(ea_l, Wconv_l, bconv_l, 1)

    hs_g = z
    hs_l = z
    for l, (f_g, f_l) in enumerate(((fg_0, fl_0), (fg_1, fl_1))):
        agg_g = _gather_mul_scatter(hs_g[0], hs_g[1], f_g, src3, dst3,
                                    zeros_block)
        agg_l = _gather_mul_scatter(hs_l[0], hs_l[1], f_l, src3, dst3,
                                    zeros_block)
        hs_g = _node_update(hs_g, agg_g, Wconv_g[l, 2], bconv_g[l, 2])
        hs_l = _node_update(hs_l, agg_l, Wconv_l[l, 2], bconv_l[l, 2])

    hh_g = _pair_products(hs_g, src3, dst3)
    hh_l = _pair_products(hs_l, src3, dst3)
    return _final_mlp(hh_g, hh_l, ea_g, ea_l,
                      Wm1_g, bm1_g, Wm2_g, bm2_g, Wm3_g, bm3_g,
                      Wm1_l, bm1_l, Wm2_l, bm2_l, Wm3_l, bm3_l)
